# baseline (speedup 1.0000x reference)
import functools

import jax
import jax.numpy as jnp
from jax import lax
from jax.experimental import pallas as pl
from jax.experimental.pallas import tpu as pltpu
from jax.experimental.pallas import tpu_sc as plsc

N = 10000
NFEAT = 128
NHID = 8
NOUT = 16
NHEADS = 8
ALPHA = 0.2

R = 20
NWROW = N // R
NPAD = 10400
BI = 400
NI = N // BI

NWORKERS = 32
WPW = 16
ROWS_W = WPW * R
EMAX = 16384
G = 128


def _take16(x, idx):
    dn = lax.GatherDimensionNumbers(
        offset_dims=(), collapsed_slice_dims=(0,), start_index_map=(0,))
    return lax.gather(x, idx[:, None], dn, (1,),
                      mode=lax.GatherScatterMode.PROMISE_IN_BOUNDS)

def _kp_body(adj_ref, out_ref):
    a3 = adj_ref[...].astype(jnp.int32).reshape(2, R, N)
    r = lax.broadcasted_iota(jnp.int32, (2, R, N), 1)
    out_ref[...] = jnp.sum(a3 << r, axis=1).reshape(1, 2, N)


def _k1_body(x_ref, wc_ref, a1_ref, a2_ref, utab_ref, rec1_ref):
    h = jnp.dot(x_ref[...], wc_ref[...], preferred_element_type=jnp.float32)
    s1 = jnp.dot(h, a1_ref[...], preferred_element_type=jnp.float32)
    s2 = jnp.dot(h, a2_ref[...], preferred_element_type=jnp.float32)
    utab_ref[...] = jnp.concatenate([jnp.exp(-s1), jnp.exp(-ALPHA * s1)], 1)
    zp = jnp.zeros((h.shape[0], 48), jnp.float32)
    rec1_ref[...] = jnp.concatenate(
        [jnp.exp(-s2), jnp.exp(-ALPHA * s2), h, zp], 1)


def _sk1_body(mask_hbm, utab_hbm, rec1_hbm,
              acc_hbm, src_hbm, dst_hbm, cnt_hbm,
              maskb, srcl, dstl, utabl, accl, gbuf, cntb, sem):
    wid = lax.axis_index("s") * 2 + lax.axis_index("c")
    wstart = wid * WPW
    wcnt = jnp.minimum(WPW, NWROW - wstart)
    rowbase = wstart * R

    iota = lax.iota(jnp.int32, 16)
    z16i = jnp.zeros((16,), jnp.int32)
    z16f = jnp.zeros((16,), jnp.float32)

    rbv = jnp.full((16,), rowbase, jnp.int32)

    def _zero_d(i, carry):
        dstl[pl.ds(i * 16, 16)] = z16i
        srcl[pl.ds(i * 16, 16)] = rbv
        return carry

    lax.fori_loop(0, EMAX // 16, _zero_d, 0)

    def _zero_a(i, carry):
        accl[pl.ds(i * 16, 16)] = z16f
        return carry

    lax.fori_loop(0, ROWS_W * 80 // 16, _zero_a, 0)

    pltpu.sync_copy(utab_hbm.at[pl.ds(rowbase * 16, ROWS_W * 16)], utabl)

    def _vec16(x):
        return x if getattr(x, "ndim", 0) == 1 else jnp.full((16,), x,
                                                             jnp.int32)

    def _row_loop(rl, curv):
        w = wstart + rl
        pltpu.sync_copy(mask_hbm.at[pl.ds(w * N, N)], maskb)
        srcbase = w * R

        def _chunk_loop(ch, curv):
            words = maskb[pl.ds(ch * 16, 16)]
            nzm0 = (words != 0).astype(jnp.int32)

            def _cond(st):
                return jnp.any(st[1] != 0)

            def _word(st):
                curv2, nzm = st
                lanev = _vec16(plsc.all_reduce_ffs(nzm != 0))
                wb = _take16(words, lanev)
                dstv = ch * 16 + lanev
                m1 = ((wb >> iota) & 1) == 1
                c1 = jnp.cumsum(m1.astype(jnp.int32))
                pos1 = curv2 + c1 - 1
                ok1 = m1 & (pos1 < EMAX)
                plsc.store_scatter(srcl, [pos1], srcbase + iota, mask=ok1)
                plsc.store_scatter(dstl, [pos1], dstv, mask=ok1)
                curv2 = curv2 + _vec16(plsc.all_reduce_population_count(ok1))
                m2 = ((((wb >> (iota + 16)) & 1) == 1) & (iota < 4))

                def _hi(cv):
                    c2 = jnp.cumsum(m2.astype(jnp.int32))
                    pos2 = cv + c2 - 1
                    ok2 = m2 & (pos2 < EMAX)
                    plsc.store_scatter(srcl, [pos2], srcbase + 16 + iota,
                                       mask=ok2)
                    plsc.store_scatter(dstl, [pos2], dstv, mask=ok2)
                    return cv + _vec16(plsc.all_reduce_population_count(ok2))

                curv2 = lax.cond(jnp.any(m2), _hi, lambda cv: cv, curv2)
                nzm2 = jnp.where(iota == lanev, 0, nzm)
                return (curv2, nzm2)

            curv, _ = lax.while_loop(_cond, _word, (curv, nzm0))
            return curv

        return lax.fori_loop(0, N // 16, _chunk_loop, curv)

    curv = lax.fori_loop(0, wcnt, _row_loop, jnp.zeros((16,), jnp.int32))
    cnt = curv[0]

    nb = (cnt + G - 1) // G

    def _batch(b, carry):
        idx = dstl.at[pl.ds(b * G, G)]
        pltpu.async_copy(rec1_hbm.at[idx], gbuf, sem).wait()
        rem = jnp.minimum(G, cnt - b * G)

        def _group(g, carry2):
            srcv = srcl[pl.ds(b * G + g * 16, 16)]
            slv = srcv - rowbase
            gm = iota < (rem - g * 16)
            rowv = g * 16 + iota
            ub = slv * 16
            ab = slv * 80
            for k in range(NHEADS):
                uk = plsc.load_gather(utabl, [ub + k])
                uak = plsc.load_gather(utabl, [ub + 8 + k])
                vk = plsc.load_gather(gbuf, [rowv, z16i + k])
                vak = plsc.load_gather(gbuf, [rowv, z16i + 8 + k])
                ek = jnp.minimum(uk * vk, uak * vak)
                plsc.addupdate_scatter(accl, [ab + 64 + k], ek, mask=gm)
                for c in range(NHID):
                    hv = plsc.load_gather(gbuf, [rowv, z16i + 16 + k * 8 + c])
                    plsc.addupdate_scatter(accl, [ab + k * 8 + c], ek * hv,
                                           mask=gm)
            return carry2

        lax.fori_loop(0, G // 16, _group, 0)
        return carry

    lax.fori_loop(0, nb, _batch, 0)

    pltpu.sync_copy(accl, acc_hbm.at[pl.ds(rowbase * 80, ROWS_W * 80)])
    pltpu.sync_copy(srcl, src_hbm.at[pl.ds(wid * EMAX, EMAX)])
    pltpu.sync_copy(dstl, dst_hbm.at[pl.ds(wid * EMAX, EMAX)])
    cntb[pl.ds(0, 16)] = jnp.full((16,), cnt, jnp.int32)
    pltpu.sync_copy(cntb, cnt_hbm.at[pl.ds(wid * 16, 16)])


def _k3_body(acc_ref, wml_ref, b1_ref, b2_ref, utab2_ref, rec2_ref):
    acc = acc_ref[...]
    hs = []
    for k in range(NHEADS):
        num = acc[:, k * NHID:(k + 1) * NHID]
        den = acc[:, 64 + k:65 + k]
        hp = num / den
        hs.append(jnp.where(hp > 0, hp, jnp.exp(hp) - 1.0))
    h1 = jnp.concatenate(hs, axis=1)
    g = jnp.dot(h1, wml_ref[...], preferred_element_type=jnp.float32)
    s1 = jnp.dot(g, b1_ref[...], preferred_element_type=jnp.float32)
    s2 = jnp.dot(g, b2_ref[...], preferred_element_type=jnp.float32)
    zpad = jnp.zeros((acc.shape[0], 12), jnp.float32)
    utab2_ref[...] = jnp.concatenate(
        [jnp.exp(-s1), jnp.exp(-ALPHA * s1), zpad], 1)
    zp48 = jnp.zeros((acc.shape[0], 80), jnp.float32)
    rec2_ref[...] = jnp.concatenate(
        [jnp.exp(-s2), jnp.exp(-ALPHA * s2), zpad, g, zp48], 1)


def _sk2_body(utab_hbm, rec2_hbm, src_hbm, dst_hbm, cnt_hbm,
              acc_hbm, srcl, dstl, utabl, accl, gbuf, cntb, sem):
    wid = lax.axis_index("s") * 2 + lax.axis_index("c")
    wstart = wid * WPW
    rowbase = wstart * R

    iota = lax.iota(jnp.int32, 16)
    z16f = jnp.zeros((16,), jnp.float32)

    def _zero_a(i, carry):
        accl[pl.ds(i * 16, 16)] = z16f
        return carry

    lax.fori_loop(0, ROWS_W * 48 // 16, _zero_a, 0)

    pltpu.sync_copy(utab_hbm.at[pl.ds(rowbase * 16, ROWS_W * 16)], utabl)
    pltpu.sync_copy(src_hbm.at[pl.ds(wid * EMAX, EMAX)], srcl)
    pltpu.sync_copy(dst_hbm.at[pl.ds(wid * EMAX, EMAX)], dstl)
    pltpu.sync_copy(cnt_hbm, cntb)
    cnt = cntb[pl.ds(wid * 16, 16)][0]
    z16i = jnp.zeros((16,), jnp.int32)

    nb = (cnt + G - 1) // G

    def _batch(b, carry):
        idx = dstl.at[pl.ds(b * G, G)]
        pltpu.async_copy(rec2_hbm.at[idx], gbuf, sem).wait()
        rem = jnp.minimum(G, cnt - b * G)

        def _group(g, carry2):
            srcv = srcl[pl.ds(b * G + g * 16, 16)]
            slv = srcv - rowbase
            gm = iota < (rem - g * 16)
            rowv = g * 16 + iota
            ub = slv * 16
            ab = slv * 48
            for t, base in ((0, 16), (1, 32)):
                ut = plsc.load_gather(utabl, [ub + t])
                uat = plsc.load_gather(utabl, [ub + 2 + t])
                vt = plsc.load_gather(gbuf, [rowv, z16i + t])
                vat = plsc.load_gather(gbuf, [rowv, z16i + 2 + t])
                et = jnp.minimum(ut * vt, uat * vat)
                plsc.addupdate_scatter(accl, [ab + t], et, mask=gm)
                for c in range(NOUT):
                    gv = plsc.load_gather(gbuf, [rowv, z16i + base + c])
                    plsc.addupdate_scatter(accl, [ab + base + c], et * gv,
                                           mask=gm)
            return carry2

        lax.fori_loop(0, G // 16, _group, 0)
        return carry

    lax.fori_loop(0, nb, _batch, 0)

    pltpu.sync_copy(accl, acc_hbm.at[pl.ds(rowbase * 48, ROWS_W * 48)])


def _k5_body(acc_ref, mu_ref, lv_ref):
    acc = acc_ref[...]
    mu_ref[...] = acc[:, 16:32] / acc[:, 0:1]
    lv_ref[...] = acc[:, 32:48] / acc[:, 1:2]


def kernel(x, adj, W, a, W_mu, a_mu, W_lv, a_lv):
    f32 = jnp.float32

    wc = jnp.transpose(W, (1, 0, 2)).reshape(NFEAT, NHEADS * NHID)
    eye = jnp.eye(NHEADS, dtype=f32)
    a1 = (a[:, 0, :NHID][:, :, None] * eye[:, None, :]).reshape(
        NHEADS * NHID, NHEADS)
    a2 = (a[:, 0, NHID:][:, :, None] * eye[:, None, :]).reshape(
        NHEADS * NHID, NHEADS)

    colmask = pl.pallas_call(
        _kp_body,
        grid=(NWROW // 2,),
        in_specs=[pl.BlockSpec((2 * R, N), lambda i: (i, 0))],
        out_specs=pl.BlockSpec((1, 2, N), lambda i: (i, 0, 0)),
        out_shape=jax.ShapeDtypeStruct((NWROW // 2, 2, N), jnp.int32),
    )(adj)
    mask_flat = colmask.reshape(NWROW * N)

    utab, rec1 = pl.pallas_call(
        _k1_body,
        grid=(NI,),
        in_specs=[
            pl.BlockSpec((BI, NFEAT), lambda i: (i, 0)),
            pl.BlockSpec((NFEAT, NHEADS * NHID), lambda i: (0, 0)),
            pl.BlockSpec((NHEADS * NHID, NHEADS), lambda i: (0, 0)),
            pl.BlockSpec((NHEADS * NHID, NHEADS), lambda i: (0, 0)),
        ],
        out_specs=[
            pl.BlockSpec((BI, 16), lambda i: (i, 0)),
            pl.BlockSpec((BI, 128), lambda i: (i, 0)),
        ],
        out_shape=[
            jax.ShapeDtypeStruct((N, 16), f32),
            jax.ShapeDtypeStruct((N, 128), f32),
        ],
    )(x, wc, a1, a2)

    utab_pad = jnp.concatenate(
        [utab, jnp.zeros((NPAD - N, 16), f32)]).reshape(NPAD * 16)

    mesh = plsc.VectorSubcoreMesh(core_axis_name="c", subcore_axis_name="s")
    sk1 = pl.kernel(
        _sk1_body, mesh=mesh,
        out_type=[
            jax.ShapeDtypeStruct((NPAD * 80,), f32),
            jax.ShapeDtypeStruct((NWORKERS * EMAX,), jnp.int32),
            jax.ShapeDtypeStruct((NWORKERS * EMAX,), jnp.int32),
            jax.ShapeDtypeStruct((NWORKERS * 16,), jnp.int32),
        ],
        scratch_types=[
            pltpu.VMEM((N,), jnp.int32),
            pltpu.VMEM((EMAX,), jnp.int32),
            pltpu.VMEM((EMAX,), jnp.int32),
            pltpu.VMEM((ROWS_W * 16,), f32),
            pltpu.VMEM((ROWS_W * 80,), f32),
            pltpu.VMEM((G, 128), f32),
            pltpu.VMEM((16,), jnp.int32),
            pltpu.SemaphoreType.DMA,
        ],
        compiler_params=pltpu.CompilerParams(needs_layout_passes=False),
    )
    acc1, srcL, dstL, cnts = sk1(mask_flat, utab_pad, rec1)

    wml = jnp.concatenate([W_mu, W_lv], axis=1)
    z2 = jnp.zeros((NOUT, 1), f32)
    b1 = jnp.concatenate([
        jnp.concatenate([a_mu[0, :NOUT, None], z2], axis=1),
        jnp.concatenate([z2, a_lv[0, :NOUT, None]], axis=1)], axis=0)
    b2 = jnp.concatenate([
        jnp.concatenate([a_mu[0, NOUT:, None], z2], axis=1),
        jnp.concatenate([z2, a_lv[0, NOUT:, None]], axis=1)], axis=0)

    utab2, rec2 = pl.pallas_call(
        _k3_body,
        grid=(NI,),
        in_specs=[
            pl.BlockSpec((BI, 80), lambda i: (i, 0)),
            pl.BlockSpec((NHEADS * NHID, 2 * NOUT), lambda i: (0, 0)),
            pl.BlockSpec((2 * NOUT, 2), lambda i: (0, 0)),
            pl.BlockSpec((2 * NOUT, 2), lambda i: (0, 0)),
        ],
        out_specs=[
            pl.BlockSpec((BI, 16), lambda i: (i, 0)),
            pl.BlockSpec((BI, 128), lambda i: (i, 0)),
        ],
        out_shape=[
            jax.ShapeDtypeStruct((N, 16), f32),
            jax.ShapeDtypeStruct((N, 128), f32),
        ],
    )(acc1.reshape(NPAD, 80)[:N], wml, b1, b2)

    utab2_pad = jnp.concatenate(
        [utab2, jnp.zeros((NPAD - N, 16), f32)]).reshape(NPAD * 16)

    sk2 = pl.kernel(
        _sk2_body, mesh=mesh,
        out_type=[jax.ShapeDtypeStruct((NPAD * 48,), f32)],
        scratch_types=[
            pltpu.VMEM((EMAX,), jnp.int32),
            pltpu.VMEM((EMAX,), jnp.int32),
            pltpu.VMEM((ROWS_W * 16,), f32),
            pltpu.VMEM((ROWS_W * 48,), f32),
            pltpu.VMEM((G, 128), f32),
            pltpu.VMEM((NWORKERS * 16,), jnp.int32),
            pltpu.SemaphoreType.DMA,
        ],
        compiler_params=pltpu.CompilerParams(needs_layout_passes=False),
    )
    acc2, = sk2(utab2_pad, rec2, srcL, dstL, cnts)

    mu, lv = pl.pallas_call(
        _k5_body,
        grid=(NI,),
        in_specs=[pl.BlockSpec((BI, 48), lambda i: (i, 0))],
        out_specs=[
            pl.BlockSpec((BI, NOUT), lambda i: (i, 0)),
            pl.BlockSpec((BI, NOUT), lambda i: (i, 0)),
        ],
        out_shape=[
            jax.ShapeDtypeStruct((N, NOUT), f32),
            jax.ShapeDtypeStruct((N, NOUT), f32),
        ],
    )(acc2.reshape(NPAD, 48)[:N])

    return (mu, mu, lv)


# speedup vs baseline: 1.2761x; 1.2761x over previous
import functools

import jax
import jax.numpy as jnp
from jax import lax
from jax.experimental import pallas as pl
from jax.experimental.pallas import tpu as pltpu
from jax.experimental.pallas import tpu_sc as plsc

N = 10000
NFEAT = 128
NHID = 8
NOUT = 16
NHEADS = 8
ALPHA = 0.2

R = 20
NWROW = N // R
NPAD = 10400
BI = 400
NI = N // BI

NWORKERS = 32
WPW = 16
ROWS_W = WPW * R
EMAX = 16384
G = 256


def _take16(x, idx):
    dn = lax.GatherDimensionNumbers(
        offset_dims=(), collapsed_slice_dims=(0,), start_index_map=(0,))
    return lax.gather(x, idx[:, None], dn, (1,),
                      mode=lax.GatherScatterMode.PROMISE_IN_BOUNDS)

def _kp_body(adj_ref, out_ref):
    a3 = adj_ref[...].astype(jnp.int32).reshape(2, R, N)
    r = lax.broadcasted_iota(jnp.int32, (2, R, N), 1)
    out_ref[...] = jnp.sum(a3 << r, axis=1).reshape(1, 2, N)


def _k1_body(x_ref, wc_ref, a1_ref, a2_ref, utab_ref, rec1_ref):
    h = jnp.dot(x_ref[...], wc_ref[...], preferred_element_type=jnp.float32)
    s1 = jnp.dot(h, a1_ref[...], preferred_element_type=jnp.float32)
    s2 = jnp.dot(h, a2_ref[...], preferred_element_type=jnp.float32)
    utab_ref[...] = jnp.concatenate([jnp.exp(-s1), jnp.exp(-ALPHA * s1)], 1)
    zp = jnp.zeros((h.shape[0], 48), jnp.float32)
    rec1_ref[...] = jnp.concatenate(
        [jnp.exp(-s2), jnp.exp(-ALPHA * s2), h, zp], 1)


def _sk1_body(mask_hbm, utab_hbm, rec1_hbm,
              acc_hbm, src_hbm, dst_hbm, cnt_hbm,
              maskb, srcl, dstl, utabl, accl, gbuf, cntb, sem):
    wid = lax.axis_index("s") * 2 + lax.axis_index("c")
    wstart = wid * WPW
    wcnt = jnp.minimum(WPW, NWROW - wstart)
    rowbase = wstart * R

    iota = lax.iota(jnp.int32, 16)
    z16i = jnp.zeros((16,), jnp.int32)
    z16f = jnp.zeros((16,), jnp.float32)

    rbv = jnp.full((16,), rowbase, jnp.int32)

    def _zero_d(i, carry):
        dstl[pl.ds(i * 16, 16)] = z16i
        srcl[pl.ds(i * 16, 16)] = rbv
        return carry

    lax.fori_loop(0, EMAX // 16, _zero_d, 0)

    def _zero_a(i, carry):
        accl[pl.ds(i * 16, 16)] = z16f
        return carry

    lax.fori_loop(0, ROWS_W * 80 // 16, _zero_a, 0)

    pltpu.sync_copy(utab_hbm.at[pl.ds(rowbase * 16, ROWS_W * 16)], utabl)

    def _vec16(x):
        return x if getattr(x, "ndim", 0) == 1 else jnp.full((16,), x,
                                                             jnp.int32)

    def _row_loop(rl, curv):
        w = wstart + rl
        pltpu.sync_copy(mask_hbm.at[pl.ds(w * N, N)], maskb)
        srcbase = w * R

        def _chunk_loop(ch, curv):
            words = maskb[pl.ds(ch * 16, 16)]
            nzm0 = (words != 0).astype(jnp.int32)

            def _cond(st):
                return jnp.any(st[1] != 0)

            def _word(st):
                curv0, nzm = st
                lanev = _vec16(plsc.all_reduce_ffs(nzm != 0))
                wb = _take16(words, lanev)
                dstv = ch * 16 + lanev
                lane0 = iota == 0

                def _single(cv):
                    expv = (lax.bitcast_convert_type(
                        wb.astype(jnp.float32), jnp.int32) >> 23) - 127
                    ok0 = lane0 & (cv < EMAX)
                    plsc.store_scatter(srcl, [cv], srcbase + expv, mask=ok0)
                    plsc.store_scatter(dstl, [cv], dstv, mask=ok0)
                    return cv + 1

                def _multi(cv):
                    m1 = ((wb >> iota) & 1) == 1
                    c1 = jnp.cumsum(m1.astype(jnp.int32))
                    pos1 = cv + c1 - 1
                    ok1 = m1 & (pos1 < EMAX)
                    plsc.store_scatter(srcl, [pos1], srcbase + iota, mask=ok1)
                    plsc.store_scatter(dstl, [pos1], dstv, mask=ok1)
                    cv = cv + _vec16(plsc.all_reduce_population_count(ok1))
                    m2 = ((((wb >> (iota + 16)) & 1) == 1) & (iota < 4))
                    c2 = jnp.cumsum(m2.astype(jnp.int32))
                    pos2 = cv + c2 - 1
                    ok2 = m2 & (pos2 < EMAX)
                    plsc.store_scatter(srcl, [pos2], srcbase + 16 + iota,
                                       mask=ok2)
                    plsc.store_scatter(dstl, [pos2], dstv, mask=ok2)
                    return cv + _vec16(plsc.all_reduce_population_count(ok2))

                curv2 = lax.cond(jnp.any((wb & (wb - 1)) == 0),
                                 _single, _multi, curv0)
                nzm2 = jnp.where(iota == lanev, 0, nzm)
                return (curv2, nzm2)

            curv, _ = lax.while_loop(_cond, _word, (curv, nzm0))
            return curv

        return lax.fori_loop(0, N // 16, _chunk_loop, curv)

    curv = lax.fori_loop(0, wcnt, _row_loop, jnp.zeros((16,), jnp.int32))
    cnt = curv[0]

    nb = (cnt + G - 1) // G

    def _batch(b, carry):
        idx = dstl.at[pl.ds(b * G, G)]
        pltpu.async_copy(rec1_hbm.at[idx], gbuf, sem).wait()
        rem = jnp.minimum(G, cnt - b * G)

        def _group(g, carry2):
            srcv = srcl[pl.ds(b * G + g * 16, 16)]
            gcnt = jnp.clip(rem - g * 16, 0, 16)

            def _edge(e16, carry3):
                sl = jnp.sum(jnp.where(iota == e16, srcv, 0)) - rowbase
                e = g * 16 + e16
                uvec = utabl[pl.ds(sl * 16, 16)]
                rv = gbuf[e, pl.ds(0, 16)]
                prod = uvec * rv
                phalf = _take16(prod, (iota + 8) & 15)
                ev = jnp.minimum(prod, phalf)
                rs = jnp.where(iota < 8, ev, 0.0)
                plsc.addupdate(accl.at[pl.ds(sl * 80 + 64, 16)], rs)
                for m in range(4):
                    hm = gbuf[e, pl.ds(16 + m * 16, 16)]
                    em = _take16(ev, (iota >> 3) + 2 * m)
                    plsc.addupdate(accl.at[pl.ds(sl * 80 + m * 16, 16)],
                                   em * hm)
                return carry3

            lax.fori_loop(0, gcnt, _edge, 0)
            return carry2

        lax.fori_loop(0, G // 16, _group, 0)
        return carry

    lax.fori_loop(0, nb, _batch, 0)

    pltpu.sync_copy(accl, acc_hbm.at[pl.ds(rowbase * 80, ROWS_W * 80)])
    pltpu.sync_copy(srcl, src_hbm.at[pl.ds(wid * EMAX, EMAX)])
    pltpu.sync_copy(dstl, dst_hbm.at[pl.ds(wid * EMAX, EMAX)])
    cntb[pl.ds(0, 16)] = jnp.full((16,), cnt, jnp.int32)
    pltpu.sync_copy(cntb, cnt_hbm.at[pl.ds(wid * 16, 16)])


def _k3_body(acc_ref, wml_ref, b1_ref, b2_ref, utab2_ref, rec2_ref):
    acc = acc_ref[...]
    hs = []
    for k in range(NHEADS):
        num = acc[:, k * NHID:(k + 1) * NHID]
        den = acc[:, 64 + k:65 + k]
        hp = num / den
        hs.append(jnp.where(hp > 0, hp, jnp.exp(hp) - 1.0))
    h1 = jnp.concatenate(hs, axis=1)
    g = jnp.dot(h1, wml_ref[...], preferred_element_type=jnp.float32)
    s1 = jnp.dot(g, b1_ref[...], preferred_element_type=jnp.float32)
    s2 = jnp.dot(g, b2_ref[...], preferred_element_type=jnp.float32)
    zpad = jnp.zeros((acc.shape[0], 12), jnp.float32)
    utab2_ref[...] = jnp.concatenate(
        [jnp.exp(-s1), jnp.exp(-ALPHA * s1), zpad], 1)
    zp48 = jnp.zeros((acc.shape[0], 80), jnp.float32)
    rec2_ref[...] = jnp.concatenate(
        [jnp.exp(-s2), jnp.exp(-ALPHA * s2), zpad, g, zp48], 1)


def _sk2_body(utab_hbm, rec2_hbm, src_hbm, dst_hbm, cnt_hbm,
              acc_hbm, srcl, dstl, utabl, accl, gbuf, cntb, sem):
    wid = lax.axis_index("s") * 2 + lax.axis_index("c")
    wstart = wid * WPW
    rowbase = wstart * R

    iota = lax.iota(jnp.int32, 16)
    z16f = jnp.zeros((16,), jnp.float32)

    def _zero_a(i, carry):
        accl[pl.ds(i * 16, 16)] = z16f
        return carry

    lax.fori_loop(0, ROWS_W * 48 // 16, _zero_a, 0)

    pltpu.sync_copy(utab_hbm.at[pl.ds(rowbase * 16, ROWS_W * 16)], utabl)
    pltpu.sync_copy(src_hbm.at[pl.ds(wid * EMAX, EMAX)], srcl)
    pltpu.sync_copy(dst_hbm.at[pl.ds(wid * EMAX, EMAX)], dstl)
    pltpu.sync_copy(cnt_hbm, cntb)
    cnt = cntb[pl.ds(wid * 16, 16)][0]
    z16i = jnp.zeros((16,), jnp.int32)

    nb = (cnt + G - 1) // G

    def _batch(b, carry):
        idx = dstl.at[pl.ds(b * G, G)]
        pltpu.async_copy(rec2_hbm.at[idx], gbuf, sem).wait()
        rem = jnp.minimum(G, cnt - b * G)

        def _group(g, carry2):
            srcv = srcl[pl.ds(b * G + g * 16, 16)]
            gcnt = jnp.clip(rem - g * 16, 0, 16)

            def _edge(e16, carry3):
                sl = jnp.sum(jnp.where(iota == e16, srcv, 0)) - rowbase
                e = g * 16 + e16
                uvec = utabl[pl.ds(sl * 16, 16)]
                rv = gbuf[e, pl.ds(0, 16)]
                prod = uvec * rv
                psh = _take16(prod, (iota + 2) & 15)
                ev = jnp.minimum(prod, psh)
                rs = jnp.where(iota < 2, ev, 0.0)
                plsc.addupdate(accl.at[pl.ds(sl * 48, 16)], rs)
                emu = _take16(ev, z16i)
                elv = _take16(ev, z16i + 1)
                gmu = gbuf[e, pl.ds(16, 16)]
                glv = gbuf[e, pl.ds(32, 16)]
                plsc.addupdate(accl.at[pl.ds(sl * 48 + 16, 16)], emu * gmu)
                plsc.addupdate(accl.at[pl.ds(sl * 48 + 32, 16)], elv * glv)
                return carry3

            lax.fori_loop(0, gcnt, _edge, 0)
            return carry2

        lax.fori_loop(0, G // 16, _group, 0)
        return carry

    lax.fori_loop(0, nb, _batch, 0)

    pltpu.sync_copy(accl, acc_hbm.at[pl.ds(rowbase * 48, ROWS_W * 48)])


def _k5_body(acc_ref, mu_ref, lv_ref):
    acc = acc_ref[...]
    mu_ref[...] = acc[:, 16:32] / acc[:, 0:1]
    lv_ref[...] = acc[:, 32:48] / acc[:, 1:2]


def kernel(x, adj, W, a, W_mu, a_mu, W_lv, a_lv):
    f32 = jnp.float32

    wc = jnp.transpose(W, (1, 0, 2)).reshape(NFEAT, NHEADS * NHID)
    eye = jnp.eye(NHEADS, dtype=f32)
    a1 = (a[:, 0, :NHID][:, :, None] * eye[:, None, :]).reshape(
        NHEADS * NHID, NHEADS)
    a2 = (a[:, 0, NHID:][:, :, None] * eye[:, None, :]).reshape(
        NHEADS * NHID, NHEADS)

    colmask = pl.pallas_call(
        _kp_body,
        grid=(NWROW // 2,),
        in_specs=[pl.BlockSpec((2 * R, N), lambda i: (i, 0))],
        out_specs=pl.BlockSpec((1, 2, N), lambda i: (i, 0, 0)),
        out_shape=jax.ShapeDtypeStruct((NWROW // 2, 2, N), jnp.int32),
    )(adj)
    mask_flat = colmask.reshape(NWROW * N)

    utab, rec1 = pl.pallas_call(
        _k1_body,
        grid=(NI,),
        in_specs=[
            pl.BlockSpec((BI, NFEAT), lambda i: (i, 0)),
            pl.BlockSpec((NFEAT, NHEADS * NHID), lambda i: (0, 0)),
            pl.BlockSpec((NHEADS * NHID, NHEADS), lambda i: (0, 0)),
            pl.BlockSpec((NHEADS * NHID, NHEADS), lambda i: (0, 0)),
        ],
        out_specs=[
            pl.BlockSpec((BI, 16), lambda i: (i, 0)),
            pl.BlockSpec((BI, 128), lambda i: (i, 0)),
        ],
        out_shape=[
            jax.ShapeDtypeStruct((NPAD, 16), f32),
            jax.ShapeDtypeStruct((N, 128), f32),
        ],
    )(x, wc, a1, a2)

    utab_pad = utab.reshape(NPAD * 16)

    mesh = plsc.VectorSubcoreMesh(core_axis_name="c", subcore_axis_name="s")
    sk1 = pl.kernel(
        _sk1_body, mesh=mesh,
        out_type=[
            jax.ShapeDtypeStruct((NPAD * 80,), f32),
            jax.ShapeDtypeStruct((NWORKERS * EMAX,), jnp.int32),
            jax.ShapeDtypeStruct((NWORKERS * EMAX,), jnp.int32),
            jax.ShapeDtypeStruct((NWORKERS * 16,), jnp.int32),
        ],
        scratch_types=[
            pltpu.VMEM((N,), jnp.int32),
            pltpu.VMEM((EMAX,), jnp.int32),
            pltpu.VMEM((EMAX,), jnp.int32),
            pltpu.VMEM((ROWS_W * 16,), f32),
            pltpu.VMEM((ROWS_W * 80,), f32),
            pltpu.VMEM((G, 128), f32),
            pltpu.VMEM((16,), jnp.int32),
            pltpu.SemaphoreType.DMA,
        ],
        compiler_params=pltpu.CompilerParams(needs_layout_passes=False),
    )
    acc1, srcL, dstL, cnts = sk1(mask_flat, utab_pad, rec1)

    wml = jnp.concatenate([W_mu, W_lv], axis=1)
    z2 = jnp.zeros((NOUT, 1), f32)
    b1 = jnp.concatenate([
        jnp.concatenate([a_mu[0, :NOUT, None], z2], axis=1),
        jnp.concatenate([z2, a_lv[0, :NOUT, None]], axis=1)], axis=0)
    b2 = jnp.concatenate([
        jnp.concatenate([a_mu[0, NOUT:, None], z2], axis=1),
        jnp.concatenate([z2, a_lv[0, NOUT:, None]], axis=1)], axis=0)

    utab2, rec2 = pl.pallas_call(
        _k3_body,
        grid=(NI,),
        in_specs=[
            pl.BlockSpec((BI, 80), lambda i: (i, 0)),
            pl.BlockSpec((NHEADS * NHID, 2 * NOUT), lambda i: (0, 0)),
            pl.BlockSpec((2 * NOUT, 2), lambda i: (0, 0)),
            pl.BlockSpec((2 * NOUT, 2), lambda i: (0, 0)),
        ],
        out_specs=[
            pl.BlockSpec((BI, 16), lambda i: (i, 0)),
            pl.BlockSpec((BI, 128), lambda i: (i, 0)),
        ],
        out_shape=[
            jax.ShapeDtypeStruct((NPAD, 16), f32),
            jax.ShapeDtypeStruct((N, 128), f32),
        ],
    )(acc1.reshape(NPAD, 80), wml, b1, b2)

    utab2_pad = utab2.reshape(NPAD * 16)

    sk2 = pl.kernel(
        _sk2_body, mesh=mesh,
        out_type=[jax.ShapeDtypeStruct((NPAD * 48,), f32)],
        scratch_types=[
            pltpu.VMEM((EMAX,), jnp.int32),
            pltpu.VMEM((EMAX,), jnp.int32),
            pltpu.VMEM((ROWS_W * 16,), f32),
            pltpu.VMEM((ROWS_W * 48,), f32),
            pltpu.VMEM((G, 128), f32),
            pltpu.VMEM((NWORKERS * 16,), jnp.int32),
            pltpu.SemaphoreType.DMA,
        ],
        compiler_params=pltpu.CompilerParams(needs_layout_passes=False),
    )
    acc2, = sk2(utab2_pad, rec2, srcL, dstL, cnts)

    mu, lv = pl.pallas_call(
        _k5_body,
        grid=(NI,),
        in_specs=[pl.BlockSpec((BI, 48), lambda i: (i, 0))],
        out_specs=[
            pl.BlockSpec((BI, NOUT), lambda i: (i, 0)),
            pl.BlockSpec((BI, NOUT), lambda i: (i, 0)),
        ],
        out_shape=[
            jax.ShapeDtypeStruct((N, NOUT), f32),
            jax.ShapeDtypeStruct((N, NOUT), f32),
        ],
    )(acc2.reshape(NPAD, 48))

    return (mu, mu, lv)


# speedup vs baseline: 1.3723x; 1.0754x over previous
import functools

import jax
import jax.numpy as jnp
from jax import lax
from jax.experimental import pallas as pl
from jax.experimental.pallas import tpu as pltpu
from jax.experimental.pallas import tpu_sc as plsc

N = 10000
NFEAT = 128
NHID = 8
NOUT = 16
NHEADS = 8
ALPHA = 0.2

R = 20
NWROW = N // R
NPAD = 10400
BI = 400
NI = N // BI

NWORKERS = 32
WPW = 16
ROWS_W = WPW * R
EMAX = 16384
G = 128


def _take16(x, idx):
    dn = lax.GatherDimensionNumbers(
        offset_dims=(), collapsed_slice_dims=(0,), start_index_map=(0,))
    return lax.gather(x, idx[:, None], dn, (1,),
                      mode=lax.GatherScatterMode.PROMISE_IN_BOUNDS)

def _kp_body(adj_ref, out_ref):
    a3 = adj_ref[...].astype(jnp.int32).reshape(2, R, N)
    r = lax.broadcasted_iota(jnp.int32, (2, R, N), 1)
    out_ref[...] = jnp.sum(a3 << r, axis=1).reshape(1, 2, N)


def _k1_body(x_ref, wc_ref, a1_ref, a2_ref, utab_ref, rec1_ref):
    h = jnp.dot(x_ref[...], wc_ref[...], preferred_element_type=jnp.float32)
    s1 = jnp.dot(h, a1_ref[...], preferred_element_type=jnp.float32)
    s2 = jnp.dot(h, a2_ref[...], preferred_element_type=jnp.float32)
    utab_ref[...] = jnp.concatenate([jnp.exp(-s1), jnp.exp(-ALPHA * s1)], 1)
    zp = jnp.zeros((h.shape[0], 48), jnp.float32)
    rec1_ref[...] = jnp.concatenate(
        [jnp.exp(-s2), jnp.exp(-ALPHA * s2), h, zp], 1)


def _sk1_body(mask_hbm, utab_hbm, rec1_hbm,
              acc_hbm, src_hbm, dst_hbm, cnt_hbm,
              maskb, srcl, dstl, utabl, accl, gbuf, cntb, sem):
    wid = lax.axis_index("s") * 2 + lax.axis_index("c")
    wstart = wid * WPW
    wcnt = jnp.minimum(WPW, NWROW - wstart)
    rowbase = wstart * R

    iota = lax.iota(jnp.int32, 16)
    z16i = jnp.zeros((16,), jnp.int32)
    z16f = jnp.zeros((16,), jnp.float32)

    def _zero_d(i, carry):
        dstl[pl.ds(i * 16, 16)] = z16i
        return carry

    lax.fori_loop(0, EMAX // 16, _zero_d, 0)

    def _zero_a(i, carry):
        accl[pl.ds(i * 16, 16)] = z16f
        return carry

    lax.fori_loop(0, ROWS_W * 80 // 16, _zero_a, 0)

    pltpu.sync_copy(utab_hbm.at[pl.ds(rowbase * 16, ROWS_W * 16)], utabl)

    def _row_loop(rl, cur):
        w = wstart + rl
        pltpu.sync_copy(mask_hbm.at[pl.ds(w * N, N)], maskb)
        srcbase = w * R

        def _chunk_loop(ch, cur):
            words = maskb[pl.ds(ch * 16, 16)]
            nzm = (words != 0).astype(jnp.int32)
            nnz = jnp.sum(nzm)

            def _cond(st):
                return st[2] > 0

            def _word(st):
                cur2, nzm, left = st
                lane = jnp.sum((jnp.cumsum(nzm) == 0).astype(jnp.int32))
                lanev = jnp.full((16,), lane, jnp.int32)
                wb_s = jnp.sum(jnp.where(iota == lanev, words, 0))
                wb = jnp.full((16,), wb_s, jnp.int32)
                dstv = jnp.full((16,), ch * 16 + lane, jnp.int32)
                m1 = ((wb >> iota) & 1) == 1
                c1 = jnp.cumsum(m1.astype(jnp.int32))
                pos1 = cur2 + c1 - 1
                ok1 = m1 & (pos1 < EMAX)
                plsc.store_scatter(srcl, [pos1], srcbase + iota, mask=ok1)
                plsc.store_scatter(dstl, [pos1], dstv, mask=ok1)
                n1 = jnp.sum(ok1.astype(jnp.int32))
                m2 = ((((wb >> (iota + 16)) & 1) == 1) & (iota < 4))
                c2 = jnp.cumsum(m2.astype(jnp.int32))
                pos2 = cur2 + n1 + c2 - 1
                ok2 = m2 & (pos2 < EMAX)
                plsc.store_scatter(srcl, [pos2], srcbase + 16 + iota, mask=ok2)
                plsc.store_scatter(dstl, [pos2], dstv, mask=ok2)
                n2 = jnp.sum(ok2.astype(jnp.int32))
                nzm2 = jnp.where(iota == lanev, 0, nzm)
                return (cur2 + n1 + n2, nzm2, left - 1)

            cur, _, _ = lax.while_loop(_cond, _word, (cur, nzm, nnz))
            return cur

        return lax.fori_loop(0, N // 16, _chunk_loop, cur)

    cnt = lax.fori_loop(0, wcnt, _row_loop, 0)

    nb = (cnt + G - 1) // G

    def _batch(b, carry):
        idx = dstl.at[pl.ds(b * G, G)]
        pltpu.async_copy(rec1_hbm.at[idx], gbuf, sem).wait()
        rem = jnp.minimum(G, cnt - b * G)

        def _group(g, carry2):
            srcv = srcl[pl.ds(b * G + g * 16, 16)]
            gcnt = jnp.clip(rem - g * 16, 0, 16)

            def _edge(e16, carry3):
                sl = jnp.sum(jnp.where(iota == e16, srcv, 0)) - rowbase
                e = g * 16 + e16
                uvec = utabl[pl.ds(sl * 16, 16)]
                rv = gbuf[e, pl.ds(0, 16)]
                prod = uvec * rv
                phalf = _take16(prod, (iota + 8) & 15)
                ev = jnp.minimum(prod, phalf)
                rs = jnp.where(iota < 8, ev, 0.0)
                plsc.addupdate(accl.at[pl.ds(sl * 80 + 64, 16)], rs)
                for m in range(4):
                    hm = gbuf[e, pl.ds(16 + m * 16, 16)]
                    em = _take16(ev, (iota >> 3) + 2 * m)
                    plsc.addupdate(accl.at[pl.ds(sl * 80 + m * 16, 16)],
                                   em * hm)
                return carry3

            lax.fori_loop(0, gcnt, _edge, 0)
            return carry2

        lax.fori_loop(0, G // 16, _group, 0)
        return carry

    lax.fori_loop(0, nb, _batch, 0)

    pltpu.sync_copy(accl, acc_hbm.at[pl.ds(rowbase * 80, ROWS_W * 80)])
    pltpu.sync_copy(srcl, src_hbm.at[pl.ds(wid * EMAX, EMAX)])
    pltpu.sync_copy(dstl, dst_hbm.at[pl.ds(wid * EMAX, EMAX)])
    cntb[pl.ds(0, 16)] = jnp.full((16,), cnt, jnp.int32)
    pltpu.sync_copy(cntb, cnt_hbm.at[pl.ds(wid * 16, 16)])


def _k3_body(acc_ref, wml_ref, b1_ref, b2_ref, utab2_ref, rec2_ref):
    acc = acc_ref[...]
    hs = []
    for k in range(NHEADS):
        num = acc[:, k * NHID:(k + 1) * NHID]
        den = acc[:, 64 + k:65 + k]
        hp = num / den
        hs.append(jnp.where(hp > 0, hp, jnp.exp(hp) - 1.0))
    h1 = jnp.concatenate(hs, axis=1)
    g = jnp.dot(h1, wml_ref[...], preferred_element_type=jnp.float32)
    s1 = jnp.dot(g, b1_ref[...], preferred_element_type=jnp.float32)
    s2 = jnp.dot(g, b2_ref[...], preferred_element_type=jnp.float32)
    zpad = jnp.zeros((acc.shape[0], 12), jnp.float32)
    utab2_ref[...] = jnp.concatenate(
        [jnp.exp(-s1), jnp.exp(-ALPHA * s1), zpad], 1)
    zp48 = jnp.zeros((acc.shape[0], 80), jnp.float32)
    rec2_ref[...] = jnp.concatenate(
        [jnp.exp(-s2), jnp.exp(-ALPHA * s2), zpad, g, zp48], 1)


def _sk2_body(utab_hbm, rec2_hbm, src_hbm, dst_hbm, cnt_hbm,
              acc_hbm, srcl, dstl, utabl, accl, gbuf, cntb, sem):
    wid = lax.axis_index("s") * 2 + lax.axis_index("c")
    wstart = wid * WPW
    rowbase = wstart * R

    iota = lax.iota(jnp.int32, 16)
    z16f = jnp.zeros((16,), jnp.float32)

    def _zero_a(i, carry):
        accl[pl.ds(i * 16, 16)] = z16f
        return carry

    lax.fori_loop(0, ROWS_W * 48 // 16, _zero_a, 0)

    pltpu.sync_copy(utab_hbm.at[pl.ds(rowbase * 16, ROWS_W * 16)], utabl)
    pltpu.sync_copy(src_hbm.at[pl.ds(wid * EMAX, EMAX)], srcl)
    pltpu.sync_copy(dst_hbm.at[pl.ds(wid * EMAX, EMAX)], dstl)
    pltpu.sync_copy(cnt_hbm, cntb)
    cnt = cntb[pl.ds(wid * 16, 16)][0]

    nb = (cnt + G - 1) // G

    def _batch(b, carry):
        idx = dstl.at[pl.ds(b * G, G)]
        pltpu.async_copy(rec2_hbm.at[idx], gbuf, sem).wait()
        rem = jnp.minimum(G, cnt - b * G)

        def _group(g, carry2):
            srcv = srcl[pl.ds(b * G + g * 16, 16)]
            gcnt = jnp.clip(rem - g * 16, 0, 16)

            def _edge(e16, carry3):
                sl = jnp.sum(jnp.where(iota == e16, srcv, 0)) - rowbase
                e = g * 16 + e16
                uvec = utabl[pl.ds(sl * 16, 16)]
                rv = gbuf[e, pl.ds(0, 16)]
                prod = uvec * rv
                psh = _take16(prod, (iota + 2) & 15)
                ev = jnp.minimum(prod, psh)
                rs = jnp.where(iota < 2, ev, 0.0)
                plsc.addupdate(accl.at[pl.ds(sl * 48, 16)], rs)
                emu = _take16(ev, jnp.zeros((16,), jnp.int32))
                elv = _take16(ev, jnp.ones((16,), jnp.int32))
                gmu = gbuf[e, pl.ds(16, 16)]
                glv = gbuf[e, pl.ds(32, 16)]
                plsc.addupdate(accl.at[pl.ds(sl * 48 + 16, 16)], emu * gmu)
                plsc.addupdate(accl.at[pl.ds(sl * 48 + 32, 16)], elv * glv)
                return carry3

            lax.fori_loop(0, gcnt, _edge, 0)
            return carry2

        lax.fori_loop(0, G // 16, _group, 0)
        return carry

    lax.fori_loop(0, nb, _batch, 0)

    pltpu.sync_copy(accl, acc_hbm.at[pl.ds(rowbase * 48, ROWS_W * 48)])


def _k5_body(acc_ref, mu_ref, lv_ref):
    acc = acc_ref[...]
    mu_ref[...] = acc[:, 16:32] / acc[:, 0:1]
    lv_ref[...] = acc[:, 32:48] / acc[:, 1:2]


def kernel(x, adj, W, a, W_mu, a_mu, W_lv, a_lv):
    f32 = jnp.float32

    wc = jnp.transpose(W, (1, 0, 2)).reshape(NFEAT, NHEADS * NHID)
    eye = jnp.eye(NHEADS, dtype=f32)
    a1 = (a[:, 0, :NHID][:, :, None] * eye[:, None, :]).reshape(
        NHEADS * NHID, NHEADS)
    a2 = (a[:, 0, NHID:][:, :, None] * eye[:, None, :]).reshape(
        NHEADS * NHID, NHEADS)

    colmask = pl.pallas_call(
        _kp_body,
        grid=(NWROW // 2,),
        in_specs=[pl.BlockSpec((2 * R, N), lambda i: (i, 0))],
        out_specs=pl.BlockSpec((1, 2, N), lambda i: (i, 0, 0)),
        out_shape=jax.ShapeDtypeStruct((NWROW // 2, 2, N), jnp.int32),
    )(adj)
    mask_flat = colmask.reshape(NWROW * N)

    utab, rec1 = pl.pallas_call(
        _k1_body,
        grid=(NI,),
        in_specs=[
            pl.BlockSpec((BI, NFEAT), lambda i: (i, 0)),
            pl.BlockSpec((NFEAT, NHEADS * NHID), lambda i: (0, 0)),
            pl.BlockSpec((NHEADS * NHID, NHEADS), lambda i: (0, 0)),
            pl.BlockSpec((NHEADS * NHID, NHEADS), lambda i: (0, 0)),
        ],
        out_specs=[
            pl.BlockSpec((BI, 16), lambda i: (i, 0)),
            pl.BlockSpec((BI, 128), lambda i: (i, 0)),
        ],
        out_shape=[
            jax.ShapeDtypeStruct((NPAD, 16), f32),
            jax.ShapeDtypeStruct((N, 128), f32),
        ],
    )(x, wc, a1, a2)

    utab_pad = utab.reshape(NPAD * 16)

    mesh = plsc.VectorSubcoreMesh(core_axis_name="c", subcore_axis_name="s")
    sk1 = pl.kernel(
        _sk1_body, mesh=mesh,
        out_type=[
            jax.ShapeDtypeStruct((NPAD * 80,), f32),
            jax.ShapeDtypeStruct((NWORKERS * EMAX,), jnp.int32),
            jax.ShapeDtypeStruct((NWORKERS * EMAX,), jnp.int32),
            jax.ShapeDtypeStruct((NWORKERS * 16,), jnp.int32),
        ],
        scratch_types=[
            pltpu.VMEM((N,), jnp.int32),
            pltpu.VMEM((EMAX,), jnp.int32),
            pltpu.VMEM((EMAX,), jnp.int32),
            pltpu.VMEM((ROWS_W * 16,), f32),
            pltpu.VMEM((ROWS_W * 80,), f32),
            pltpu.VMEM((G, 128), f32),
            pltpu.VMEM((16,), jnp.int32),
            pltpu.SemaphoreType.DMA,
        ],
        compiler_params=pltpu.CompilerParams(needs_layout_passes=False),
    )
    acc1, srcL, dstL, cnts = sk1(mask_flat, utab_pad, rec1)

    wml = jnp.concatenate([W_mu, W_lv], axis=1)
    z2 = jnp.zeros((NOUT, 1), f32)
    b1 = jnp.concatenate([
        jnp.concatenate([a_mu[0, :NOUT, None], z2], axis=1),
        jnp.concatenate([z2, a_lv[0, :NOUT, None]], axis=1)], axis=0)
    b2 = jnp.concatenate([
        jnp.concatenate([a_mu[0, NOUT:, None], z2], axis=1),
        jnp.concatenate([z2, a_lv[0, NOUT:, None]], axis=1)], axis=0)

    utab2, rec2 = pl.pallas_call(
        _k3_body,
        grid=(NI,),
        in_specs=[
            pl.BlockSpec((BI, 80), lambda i: (i, 0)),
            pl.BlockSpec((NHEADS * NHID, 2 * NOUT), lambda i: (0, 0)),
            pl.BlockSpec((2 * NOUT, 2), lambda i: (0, 0)),
            pl.BlockSpec((2 * NOUT, 2), lambda i: (0, 0)),
        ],
        out_specs=[
            pl.BlockSpec((BI, 16), lambda i: (i, 0)),
            pl.BlockSpec((BI, 128), lambda i: (i, 0)),
        ],
        out_shape=[
            jax.ShapeDtypeStruct((NPAD, 16), f32),
            jax.ShapeDtypeStruct((N, 128), f32),
        ],
    )(acc1.reshape(NPAD, 80), wml, b1, b2)

    utab2_pad = utab2.reshape(NPAD * 16)

    sk2 = pl.kernel(
        _sk2_body, mesh=mesh,
        out_type=[jax.ShapeDtypeStruct((NPAD * 48,), f32)],
        scratch_types=[
            pltpu.VMEM((EMAX,), jnp.int32),
            pltpu.VMEM((EMAX,), jnp.int32),
            pltpu.VMEM((ROWS_W * 16,), f32),
            pltpu.VMEM((ROWS_W * 48,), f32),
            pltpu.VMEM((G, 128), f32),
            pltpu.VMEM((NWORKERS * 16,), jnp.int32),
            pltpu.SemaphoreType.DMA,
        ],
        compiler_params=pltpu.CompilerParams(needs_layout_passes=False),
    )
    acc2, = sk2(utab2_pad, rec2, srcL, dstL, cnts)

    mu, lv = pl.pallas_call(
        _k5_body,
        grid=(NI,),
        in_specs=[pl.BlockSpec((BI, 48), lambda i: (i, 0))],
        out_specs=[
            pl.BlockSpec((BI, NOUT), lambda i: (i, 0)),
            pl.BlockSpec((BI, NOUT), lambda i: (i, 0)),
        ],
        out_shape=[
            jax.ShapeDtypeStruct((N, NOUT), f32),
            jax.ShapeDtypeStruct((N, NOUT), f32),
        ],
    )(acc2.reshape(NPAD, 48))

    return (mu, mu, lv)


# speedup vs baseline: 1.3743x; 1.0014x over previous
import functools

import jax
import jax.numpy as jnp
from jax import lax
from jax.experimental import pallas as pl
from jax.experimental.pallas import tpu as pltpu
from jax.experimental.pallas import tpu_sc as plsc

N = 10000
NFEAT = 128
NHID = 8
NOUT = 16
NHEADS = 8
ALPHA = 0.2

R = 20
NWROW = N // R
NPAD = 10400
BI = 400
NI = N // BI

NWORKERS = 32
WPW = 16
ROWS_W = WPW * R
EMAX = 16384
G = 128


def _take16(x, idx):
    dn = lax.GatherDimensionNumbers(
        offset_dims=(), collapsed_slice_dims=(0,), start_index_map=(0,))
    return lax.gather(x, idx[:, None], dn, (1,),
                      mode=lax.GatherScatterMode.PROMISE_IN_BOUNDS)

def _kp_body(adj_ref, out_ref):
    a3 = adj_ref[...].astype(jnp.int32).reshape(2, R, N)
    r = lax.broadcasted_iota(jnp.int32, (2, R, N), 1)
    out_ref[...] = jnp.sum(a3 << r, axis=1).reshape(1, 2, N)


def _k1_body(x_ref, wc_ref, a1_ref, a2_ref, utab_ref, rec1_ref):
    h = jnp.dot(x_ref[...], wc_ref[...], preferred_element_type=jnp.float32)
    s1 = jnp.dot(h, a1_ref[...], preferred_element_type=jnp.float32)
    s2 = jnp.dot(h, a2_ref[...], preferred_element_type=jnp.float32)
    utab_ref[...] = jnp.concatenate([jnp.exp(-s1), jnp.exp(-ALPHA * s1)], 1)
    zp = jnp.zeros((h.shape[0], 48), jnp.float32)
    rec1_ref[...] = jnp.concatenate(
        [jnp.exp(-s2), jnp.exp(-ALPHA * s2), h, zp], 1)


def _sk1_body(mask_hbm, utab_hbm, rec1_hbm,
              acc_hbm, src_hbm, dst_hbm, cnt_hbm,
              maskb, srcl, dstl, utabl, accl, gbuf, cntb, sem):
    wid = lax.axis_index("s") * 2 + lax.axis_index("c")
    wstart = wid * WPW
    wcnt = jnp.minimum(WPW, NWROW - wstart)
    rowbase = wstart * R

    iota = lax.iota(jnp.int32, 16)
    z16i = jnp.zeros((16,), jnp.int32)
    z16f = jnp.zeros((16,), jnp.float32)

    def _zero_d(i, carry):
        dstl[pl.ds(i * 16, 16)] = z16i
        return carry

    lax.fori_loop(0, EMAX // 16, _zero_d, 0)

    def _zero_a(i, carry):
        accl[pl.ds(i * 16, 16)] = z16f
        return carry

    lax.fori_loop(0, ROWS_W * 80 // 16, _zero_a, 0)

    pltpu.sync_copy(utab_hbm.at[pl.ds(rowbase * 16, ROWS_W * 16)], utabl)

    def _row_loop(rl, cur):
        w = wstart + rl
        pltpu.sync_copy(mask_hbm.at[pl.ds(w * N, N)], maskb)
        srcbase = w * R

        def _chunk_loop(ch, cur):
            words = maskb[pl.ds(ch * 16, 16)]
            nzm0 = (words != 0).astype(jnp.int32)

            def _cond(st):
                return jnp.any(st[1] != 0)

            def _word(st):
                cur2, nzm = st
                lanev = plsc.all_reduce_ffs(nzm != 0)
                if getattr(lanev, "ndim", 0) == 0:
                    lanev = jnp.full((16,), lanev, jnp.int32)
                wb = _take16(words, lanev)
                dstv = ch * 16 + lanev
                m1 = ((wb >> iota) & 1) == 1
                c1 = jnp.cumsum(m1.astype(jnp.int32))
                pos1 = cur2 + c1 - 1
                ok1 = m1 & (pos1 < EMAX)
                plsc.store_scatter(srcl, [pos1], srcbase + iota, mask=ok1)
                plsc.store_scatter(dstl, [pos1], dstv, mask=ok1)
                n1 = jnp.sum(ok1.astype(jnp.int32))
                m2 = ((((wb >> (iota + 16)) & 1) == 1) & (iota < 4))
                c2 = jnp.cumsum(m2.astype(jnp.int32))
                pos2 = cur2 + n1 + c2 - 1
                ok2 = m2 & (pos2 < EMAX)
                plsc.store_scatter(srcl, [pos2], srcbase + 16 + iota, mask=ok2)
                plsc.store_scatter(dstl, [pos2], dstv, mask=ok2)
                n2 = jnp.sum(ok2.astype(jnp.int32))
                nzm2 = jnp.where(iota == lanev, 0, nzm)
                return (cur2 + n1 + n2, nzm2)

            cur, _ = lax.while_loop(_cond, _word, (cur, nzm0))
            return cur

        return lax.fori_loop(0, N // 16, _chunk_loop, cur)

    cnt = lax.fori_loop(0, wcnt, _row_loop, 0)

    nb = (cnt + G - 1) // G

    def _batch(b, carry):
        idx = dstl.at[pl.ds(b * G, G)]
        pltpu.async_copy(rec1_hbm.at[idx], gbuf, sem).wait()
        rem = jnp.minimum(G, cnt - b * G)

        def _group(g, carry2):
            srcv = srcl[pl.ds(b * G + g * 16, 16)]
            gcnt = jnp.clip(rem - g * 16, 0, 16)

            def _edge(e16, carry3):
                sl = jnp.sum(jnp.where(iota == e16, srcv, 0)) - rowbase
                e = g * 16 + e16
                uvec = utabl[pl.ds(sl * 16, 16)]
                rv = gbuf[e, pl.ds(0, 16)]
                prod = uvec * rv
                phalf = _take16(prod, (iota + 8) & 15)
                ev = jnp.minimum(prod, phalf)
                rs = jnp.where(iota < 8, ev, 0.0)
                plsc.addupdate(accl.at[pl.ds(sl * 80 + 64, 16)], rs)
                for m in range(4):
                    hm = gbuf[e, pl.ds(16 + m * 16, 16)]
                    em = _take16(ev, (iota >> 3) + 2 * m)
                    plsc.addupdate(accl.at[pl.ds(sl * 80 + m * 16, 16)],
                                   em * hm)
                return carry3

            lax.fori_loop(0, gcnt, _edge, 0)
            return carry2

        lax.fori_loop(0, G // 16, _group, 0)
        return carry

    lax.fori_loop(0, nb, _batch, 0)

    pltpu.sync_copy(accl, acc_hbm.at[pl.ds(rowbase * 80, ROWS_W * 80)])
    pltpu.sync_copy(srcl, src_hbm.at[pl.ds(wid * EMAX, EMAX)])
    pltpu.sync_copy(dstl, dst_hbm.at[pl.ds(wid * EMAX, EMAX)])
    cntb[pl.ds(0, 16)] = jnp.full((16,), cnt, jnp.int32)
    pltpu.sync_copy(cntb, cnt_hbm.at[pl.ds(wid * 16, 16)])


def _k3_body(acc_ref, wml_ref, b1_ref, b2_ref, utab2_ref, rec2_ref):
    acc = acc_ref[...]
    hs = []
    for k in range(NHEADS):
        num = acc[:, k * NHID:(k + 1) * NHID]
        den = acc[:, 64 + k:65 + k]
        hp = num / den
        hs.append(jnp.where(hp > 0, hp, jnp.exp(hp) - 1.0))
    h1 = jnp.concatenate(hs, axis=1)
    g = jnp.dot(h1, wml_ref[...], preferred_element_type=jnp.float32)
    s1 = jnp.dot(g, b1_ref[...], preferred_element_type=jnp.float32)
    s2 = jnp.dot(g, b2_ref[...], preferred_element_type=jnp.float32)
    zpad = jnp.zeros((acc.shape[0], 12), jnp.float32)
    utab2_ref[...] = jnp.concatenate(
        [jnp.exp(-s1), jnp.exp(-ALPHA * s1), zpad], 1)
    zp48 = jnp.zeros((acc.shape[0], 80), jnp.float32)
    rec2_ref[...] = jnp.concatenate(
        [jnp.exp(-s2), jnp.exp(-ALPHA * s2), zpad, g, zp48], 1)


def _sk2_body(utab_hbm, rec2_hbm, src_hbm, dst_hbm, cnt_hbm,
              acc_hbm, srcl, dstl, utabl, accl, gbuf, cntb, sem):
    wid = lax.axis_index("s") * 2 + lax.axis_index("c")
    wstart = wid * WPW
    rowbase = wstart * R

    iota = lax.iota(jnp.int32, 16)
    z16f = jnp.zeros((16,), jnp.float32)

    def _zero_a(i, carry):
        accl[pl.ds(i * 16, 16)] = z16f
        return carry

    lax.fori_loop(0, ROWS_W * 48 // 16, _zero_a, 0)

    pltpu.sync_copy(utab_hbm.at[pl.ds(rowbase * 16, ROWS_W * 16)], utabl)
    pltpu.sync_copy(src_hbm.at[pl.ds(wid * EMAX, EMAX)], srcl)
    pltpu.sync_copy(dst_hbm.at[pl.ds(wid * EMAX, EMAX)], dstl)
    pltpu.sync_copy(cnt_hbm, cntb)
    cnt = cntb[pl.ds(wid * 16, 16)][0]

    nb = (cnt + G - 1) // G

    def _batch(b, carry):
        idx = dstl.at[pl.ds(b * G, G)]
        pltpu.async_copy(rec2_hbm.at[idx], gbuf, sem).wait()
        rem = jnp.minimum(G, cnt - b * G)

        def _group(g, carry2):
            srcv = srcl[pl.ds(b * G + g * 16, 16)]
            gcnt = jnp.clip(rem - g * 16, 0, 16)

            def _edge(e16, carry3):
                sl = jnp.sum(jnp.where(iota == e16, srcv, 0)) - rowbase
                e = g * 16 + e16
                uvec = utabl[pl.ds(sl * 16, 16)]
                rv = gbuf[e, pl.ds(0, 16)]
                prod = uvec * rv
                psh = _take16(prod, (iota + 2) & 15)
                ev = jnp.minimum(prod, psh)
                rs = jnp.where(iota < 2, ev, 0.0)
                plsc.addupdate(accl.at[pl.ds(sl * 48, 16)], rs)
                emu = _take16(ev, jnp.zeros((16,), jnp.int32))
                elv = _take16(ev, jnp.ones((16,), jnp.int32))
                gmu = gbuf[e, pl.ds(16, 16)]
                glv = gbuf[e, pl.ds(32, 16)]
                plsc.addupdate(accl.at[pl.ds(sl * 48 + 16, 16)], emu * gmu)
                plsc.addupdate(accl.at[pl.ds(sl * 48 + 32, 16)], elv * glv)
                return carry3

            lax.fori_loop(0, gcnt, _edge, 0)
            return carry2

        lax.fori_loop(0, G // 16, _group, 0)
        return carry

    lax.fori_loop(0, nb, _batch, 0)

    pltpu.sync_copy(accl, acc_hbm.at[pl.ds(rowbase * 48, ROWS_W * 48)])


def _k5_body(acc_ref, mu_ref, lv_ref):
    acc = acc_ref[...]
    mu_ref[...] = acc[:, 16:32] / acc[:, 0:1]
    lv_ref[...] = acc[:, 32:48] / acc[:, 1:2]


def kernel(x, adj, W, a, W_mu, a_mu, W_lv, a_lv):
    f32 = jnp.float32

    wc = jnp.transpose(W, (1, 0, 2)).reshape(NFEAT, NHEADS * NHID)
    eye = jnp.eye(NHEADS, dtype=f32)
    a1 = (a[:, 0, :NHID][:, :, None] * eye[:, None, :]).reshape(
        NHEADS * NHID, NHEADS)
    a2 = (a[:, 0, NHID:][:, :, None] * eye[:, None, :]).reshape(
        NHEADS * NHID, NHEADS)

    colmask = pl.pallas_call(
        _kp_body,
        grid=(NWROW // 2,),
        in_specs=[pl.BlockSpec((2 * R, N), lambda i: (i, 0))],
        out_specs=pl.BlockSpec((1, 2, N), lambda i: (i, 0, 0)),
        out_shape=jax.ShapeDtypeStruct((NWROW // 2, 2, N), jnp.int32),
    )(adj)
    mask_flat = colmask.reshape(NWROW * N)

    utab, rec1 = pl.pallas_call(
        _k1_body,
        grid=(NI,),
        in_specs=[
            pl.BlockSpec((BI, NFEAT), lambda i: (i, 0)),
            pl.BlockSpec((NFEAT, NHEADS * NHID), lambda i: (0, 0)),
            pl.BlockSpec((NHEADS * NHID, NHEADS), lambda i: (0, 0)),
            pl.BlockSpec((NHEADS * NHID, NHEADS), lambda i: (0, 0)),
        ],
        out_specs=[
            pl.BlockSpec((BI, 16), lambda i: (i, 0)),
            pl.BlockSpec((BI, 128), lambda i: (i, 0)),
        ],
        out_shape=[
            jax.ShapeDtypeStruct((NPAD, 16), f32),
            jax.ShapeDtypeStruct((N, 128), f32),
        ],
    )(x, wc, a1, a2)

    utab_pad = utab.reshape(NPAD * 16)

    mesh = plsc.VectorSubcoreMesh(core_axis_name="c", subcore_axis_name="s")
    sk1 = pl.kernel(
        _sk1_body, mesh=mesh,
        out_type=[
            jax.ShapeDtypeStruct((NPAD * 80,), f32),
            jax.ShapeDtypeStruct((NWORKERS * EMAX,), jnp.int32),
            jax.ShapeDtypeStruct((NWORKERS * EMAX,), jnp.int32),
            jax.ShapeDtypeStruct((NWORKERS * 16,), jnp.int32),
        ],
        scratch_types=[
            pltpu.VMEM((N,), jnp.int32),
            pltpu.VMEM((EMAX,), jnp.int32),
            pltpu.VMEM((EMAX,), jnp.int32),
            pltpu.VMEM((ROWS_W * 16,), f32),
            pltpu.VMEM((ROWS_W * 80,), f32),
            pltpu.VMEM((G, 128), f32),
            pltpu.VMEM((16,), jnp.int32),
            pltpu.SemaphoreType.DMA,
        ],
        compiler_params=pltpu.CompilerParams(needs_layout_passes=False),
    )
    acc1, srcL, dstL, cnts = sk1(mask_flat, utab_pad, rec1)

    wml = jnp.concatenate([W_mu, W_lv], axis=1)
    z2 = jnp.zeros((NOUT, 1), f32)
    b1 = jnp.concatenate([
        jnp.concatenate([a_mu[0, :NOUT, None], z2], axis=1),
        jnp.concatenate([z2, a_lv[0, :NOUT, None]], axis=1)], axis=0)
    b2 = jnp.concatenate([
        jnp.concatenate([a_mu[0, NOUT:, None], z2], axis=1),
        jnp.concatenate([z2, a_lv[0, NOUT:, None]], axis=1)], axis=0)

    utab2, rec2 = pl.pallas_call(
        _k3_body,
        grid=(NI,),
        in_specs=[
            pl.BlockSpec((BI, 80), lambda i: (i, 0)),
            pl.BlockSpec((NHEADS * NHID, 2 * NOUT), lambda i: (0, 0)),
            pl.BlockSpec((2 * NOUT, 2), lambda i: (0, 0)),
            pl.BlockSpec((2 * NOUT, 2), lambda i: (0, 0)),
        ],
        out_specs=[
            pl.BlockSpec((BI, 16), lambda i: (i, 0)),
            pl.BlockSpec((BI, 128), lambda i: (i, 0)),
        ],
        out_shape=[
            jax.ShapeDtypeStruct((NPAD, 16), f32),
            jax.ShapeDtypeStruct((N, 128), f32),
        ],
    )(acc1.reshape(NPAD, 80), wml, b1, b2)

    utab2_pad = utab2.reshape(NPAD * 16)

    sk2 = pl.kernel(
        _sk2_body, mesh=mesh,
        out_type=[jax.ShapeDtypeStruct((NPAD * 48,), f32)],
        scratch_types=[
            pltpu.VMEM((EMAX,), jnp.int32),
            pltpu.VMEM((EMAX,), jnp.int32),
            pltpu.VMEM((ROWS_W * 16,), f32),
            pltpu.VMEM((ROWS_W * 48,), f32),
            pltpu.VMEM((G, 128), f32),
            pltpu.VMEM((NWORKERS * 16,), jnp.int32),
            pltpu.SemaphoreType.DMA,
        ],
        compiler_params=pltpu.CompilerParams(needs_layout_passes=False),
    )
    acc2, = sk2(utab2_pad, rec2, srcL, dstL, cnts)

    mu, lv = pl.pallas_call(
        _k5_body,
        grid=(NI,),
        in_specs=[pl.BlockSpec((BI, 48), lambda i: (i, 0))],
        out_specs=[
            pl.BlockSpec((BI, NOUT), lambda i: (i, 0)),
            pl.BlockSpec((BI, NOUT), lambda i: (i, 0)),
        ],
        out_shape=[
            jax.ShapeDtypeStruct((N, NOUT), f32),
            jax.ShapeDtypeStruct((N, NOUT), f32),
        ],
    )(acc2.reshape(NPAD, 48))

    return (mu, mu, lv)


# speedup vs baseline: 1.3746x; 1.0002x over previous
import jax
import jax.numpy as jnp
from jax import lax
from jax.experimental import pallas as pl
from jax.experimental.pallas import tpu as pltpu
from jax.experimental.pallas import tpu_sc as plsc

N = 10000
NFEAT = 128
NHID = 8
NOUT = 16
NHEADS = 8
ALPHA = 0.2

R = 20
NWROW = N // R
NPAD = 10400
BI = 400
NI = N // BI

NWORKERS = 32
WPW = 16
ROWS_W = WPW * R
EMAX = 16384
G = 128


def _take16(x, idx):
    dn = lax.GatherDimensionNumbers(
        offset_dims=(), collapsed_slice_dims=(0,), start_index_map=(0,))
    return lax.gather(x, idx[:, None], dn, (1,),
                      mode=lax.GatherScatterMode.PROMISE_IN_BOUNDS)

def _kp_body(adj_ref, out_ref):
    a3 = adj_ref[...].astype(jnp.int32).reshape(2, R, N)
    r = lax.broadcasted_iota(jnp.int32, (2, R, N), 1)
    out_ref[...] = jnp.sum(a3 << r, axis=1).reshape(1, 2, N)


def _k1_body(x_ref, wc_ref, a1_ref, a2_ref, utab_ref, rec1_ref):
    h = jnp.dot(x_ref[...], wc_ref[...], preferred_element_type=jnp.float32)
    s1 = jnp.dot(h, a1_ref[...], preferred_element_type=jnp.float32)
    s2 = jnp.dot(h, a2_ref[...], preferred_element_type=jnp.float32)
    utab_ref[...] = jnp.concatenate([jnp.exp(-s1), jnp.exp(-ALPHA * s1)], 1)
    zp = jnp.zeros((h.shape[0], 48), jnp.float32)
    rec1_ref[...] = jnp.concatenate(
        [jnp.exp(-s2), jnp.exp(-ALPHA * s2), h, zp], 1)


def _sk1_body(mask_hbm, utab_hbm, rec1_hbm,
              acc_hbm, src_hbm, dst_hbm, cnt_hbm,
              maskb, srcl, dstl, utabl, accl, gbuf, cntb, sem):
    wid = lax.axis_index("s") * 2 + lax.axis_index("c")
    wstart = wid * WPW
    wcnt = jnp.minimum(WPW, NWROW - wstart)
    rowbase = wstart * R

    iota = lax.iota(jnp.int32, 16)
    z16i = jnp.zeros((16,), jnp.int32)
    z16f = jnp.zeros((16,), jnp.float32)

    def _zero_d(i, carry):
        dstl[pl.ds(i * 16, 16)] = z16i
        return carry

    lax.fori_loop(0, EMAX // 16, _zero_d, 0)

    def _zero_a(i, carry):
        accl[pl.ds(i * 16, 16)] = z16f
        return carry

    lax.fori_loop(0, ROWS_W * 80 // 16, _zero_a, 0)

    pltpu.sync_copy(utab_hbm.at[pl.ds(rowbase * 16, ROWS_W * 16)], utabl)

    def _row_loop(rl, cur):
        w = wstart + rl
        pltpu.sync_copy(mask_hbm.at[pl.ds(w * N, N)], maskb)
        srcbase = w * R

        def _chunk_loop(ch, cur):
            words = maskb[pl.ds(ch * 16, 16)]
            nzm0 = (words != 0).astype(jnp.int32)

            def _cond(st):
                return jnp.any(st[1] != 0)

            def _word(st):
                cur2, nzm = st
                lanev = plsc.all_reduce_ffs(nzm != 0)
                if getattr(lanev, "ndim", 0) == 0:
                    lanev = jnp.full((16,), lanev, jnp.int32)
                wb = _take16(words, lanev)
                dstv = ch * 16 + lanev
                m1 = ((wb >> iota) & 1) == 1
                c1 = jnp.cumsum(m1.astype(jnp.int32))
                pos1 = cur2 + c1 - 1
                ok1 = m1 & (pos1 < EMAX)
                plsc.store_scatter(srcl, [pos1], srcbase + iota, mask=ok1)
                plsc.store_scatter(dstl, [pos1], dstv, mask=ok1)
                n1 = jnp.sum(ok1.astype(jnp.int32))
                m2 = ((((wb >> (iota + 16)) & 1) == 1) & (iota < 4))
                c2 = jnp.cumsum(m2.astype(jnp.int32))
                pos2 = cur2 + n1 + c2 - 1
                ok2 = m2 & (pos2 < EMAX)
                plsc.store_scatter(srcl, [pos2], srcbase + 16 + iota, mask=ok2)
                plsc.store_scatter(dstl, [pos2], dstv, mask=ok2)
                n2 = jnp.sum(ok2.astype(jnp.int32))
                nzm2 = jnp.where(iota == lanev, 0, nzm)
                return (cur2 + n1 + n2, nzm2)

            cur, _ = lax.while_loop(_cond, _word, (cur, nzm0))
            return cur

        return lax.fori_loop(0, N // 16, _chunk_loop, cur)

    cnt = lax.fori_loop(0, wcnt, _row_loop, 0)

    nb = (cnt + G - 1) // G

    def _batch(b, carry):
        idx = dstl.at[pl.ds(b * G, G)]
        pltpu.async_copy(rec1_hbm.at[idx], gbuf, sem).wait()
        rem = jnp.minimum(G, cnt - b * G)

        def _group(g, carry2):
            srcv = srcl[pl.ds(b * G + g * 16, 16)]
            gcnt = jnp.clip(rem - g * 16, 0, 16)

            def _edge(e16, carry3):
                sl = jnp.sum(jnp.where(iota == e16, srcv, 0)) - rowbase
                e = g * 16 + e16
                uvec = utabl[pl.ds(sl * 16, 16)]
                rv = gbuf[e, pl.ds(0, 16)]
                prod = uvec * rv
                phalf = _take16(prod, (iota + 8) & 15)
                ev = jnp.minimum(prod, phalf)
                rs = jnp.where(iota < 8, ev, 0.0)
                plsc.addupdate(accl.at[pl.ds(sl * 80 + 64, 16)], rs)
                for m in range(4):
                    hm = gbuf[e, pl.ds(16 + m * 16, 16)]
                    em = _take16(ev, (iota >> 3) + 2 * m)
                    plsc.addupdate(accl.at[pl.ds(sl * 80 + m * 16, 16)],
                                   em * hm)
                return carry3

            lax.fori_loop(0, gcnt, _edge, 0)
            return carry2

        lax.fori_loop(0, G // 16, _group, 0)
        return carry

    lax.fori_loop(0, nb, _batch, 0)

    pltpu.sync_copy(accl, acc_hbm.at[pl.ds(rowbase * 80, ROWS_W * 80)])
    pltpu.sync_copy(srcl, src_hbm.at[pl.ds(wid * EMAX, EMAX)])
    pltpu.sync_copy(dstl, dst_hbm.at[pl.ds(wid * EMAX, EMAX)])
    cntb[pl.ds(0, 16)] = jnp.full((16,), cnt, jnp.int32)
    pltpu.sync_copy(cntb, cnt_hbm.at[pl.ds(wid * 16, 16)])


def _k3_body(acc_ref, wml_ref, b1_ref, b2_ref, utab2_ref, rec2_ref):
    acc = acc_ref[...]
    hs = []
    for k in range(NHEADS):
        num = acc[:, k * NHID:(k + 1) * NHID]
        den = acc[:, 64 + k:65 + k]
        hp = num / den
        hs.append(jnp.where(hp > 0, hp, jnp.exp(hp) - 1.0))
    h1 = jnp.concatenate(hs, axis=1)
    g = jnp.dot(h1, wml_ref[...], preferred_element_type=jnp.float32)
    s1 = jnp.dot(g, b1_ref[...], preferred_element_type=jnp.float32)
    s2 = jnp.dot(g, b2_ref[...], preferred_element_type=jnp.float32)
    zpad = jnp.zeros((acc.shape[0], 12), jnp.float32)
    utab2_ref[...] = jnp.concatenate(
        [jnp.exp(-s1), jnp.exp(-ALPHA * s1), zpad], 1)
    zp48 = jnp.zeros((acc.shape[0], 80), jnp.float32)
    rec2_ref[...] = jnp.concatenate(
        [jnp.exp(-s2), jnp.exp(-ALPHA * s2), zpad, g, zp48], 1)


def _sk2_body(utab_hbm, rec2_hbm, src_hbm, dst_hbm, cnt_hbm,
              acc_hbm, srcl, dstl, utabl, accl, gbuf, cntb, sem):
    wid = lax.axis_index("s") * 2 + lax.axis_index("c")
    wstart = wid * WPW
    rowbase = wstart * R

    iota = lax.iota(jnp.int32, 16)
    z16f = jnp.zeros((16,), jnp.float32)

    def _zero_a(i, carry):
        accl[pl.ds(i * 16, 16)] = z16f
        return carry

    lax.fori_loop(0, ROWS_W * 48 // 16, _zero_a, 0)

    pltpu.sync_copy(utab_hbm.at[pl.ds(rowbase * 16, ROWS_W * 16)], utabl)
    pltpu.sync_copy(src_hbm.at[pl.ds(wid * EMAX, EMAX)], srcl)
    pltpu.sync_copy(dst_hbm.at[pl.ds(wid * EMAX, EMAX)], dstl)
    pltpu.sync_copy(cnt_hbm, cntb)
    cnt = cntb[pl.ds(wid * 16, 16)][0]

    nb = (cnt + G - 1) // G

    def _batch(b, carry):
        idx = dstl.at[pl.ds(b * G, G)]
        pltpu.async_copy(rec2_hbm.at[idx], gbuf, sem).wait()
        rem = jnp.minimum(G, cnt - b * G)

        def _group(g, carry2):
            srcv = srcl[pl.ds(b * G + g * 16, 16)]
            gcnt = jnp.clip(rem - g * 16, 0, 16)

            def _edge(e16, carry3):
                sl = jnp.sum(jnp.where(iota == e16, srcv, 0)) - rowbase
                e = g * 16 + e16
                uvec = utabl[pl.ds(sl * 16, 16)]
                rv = gbuf[e, pl.ds(0, 16)]
                prod = uvec * rv
                psh = _take16(prod, (iota + 2) & 15)
                ev = jnp.minimum(prod, psh)
                rs = jnp.where(iota < 2, ev, 0.0)
                plsc.addupdate(accl.at[pl.ds(sl * 48, 16)], rs)
                emu = _take16(ev, jnp.zeros((16,), jnp.int32))
                elv = _take16(ev, jnp.ones((16,), jnp.int32))
                gmu = gbuf[e, pl.ds(16, 16)]
                glv = gbuf[e, pl.ds(32, 16)]
                plsc.addupdate(accl.at[pl.ds(sl * 48 + 16, 16)], emu * gmu)
                plsc.addupdate(accl.at[pl.ds(sl * 48 + 32, 16)], elv * glv)
                return carry3

            lax.fori_loop(0, gcnt, _edge, 0)
            return carry2

        lax.fori_loop(0, G // 16, _group, 0)
        return carry

    lax.fori_loop(0, nb, _batch, 0)

    pltpu.sync_copy(accl, acc_hbm.at[pl.ds(rowbase * 48, ROWS_W * 48)])


def _k5_body(acc_ref, mu_ref, lv_ref):
    acc = acc_ref[...]
    mu_ref[...] = acc[:, 16:32] / acc[:, 0:1]
    lv_ref[...] = acc[:, 32:48] / acc[:, 1:2]


def kernel(x, adj, W, a, W_mu, a_mu, W_lv, a_lv):
    f32 = jnp.float32

    wc = jnp.transpose(W, (1, 0, 2)).reshape(NFEAT, NHEADS * NHID)
    eye = jnp.eye(NHEADS, dtype=f32)
    a1 = (a[:, 0, :NHID][:, :, None] * eye[:, None, :]).reshape(
        NHEADS * NHID, NHEADS)
    a2 = (a[:, 0, NHID:][:, :, None] * eye[:, None, :]).reshape(
        NHEADS * NHID, NHEADS)

    colmask = pl.pallas_call(
        _kp_body,
        grid=(NWROW // 2,),
        in_specs=[pl.BlockSpec((2 * R, N), lambda i: (i, 0))],
        out_specs=pl.BlockSpec((1, 2, N), lambda i: (i, 0, 0)),
        out_shape=jax.ShapeDtypeStruct((NWROW // 2, 2, N), jnp.int32),
    )(adj)
    mask_flat = colmask.reshape(NWROW * N)

    utab, rec1 = pl.pallas_call(
        _k1_body,
        grid=(NI,),
        in_specs=[
            pl.BlockSpec((BI, NFEAT), lambda i: (i, 0)),
            pl.BlockSpec((NFEAT, NHEADS * NHID), lambda i: (0, 0)),
            pl.BlockSpec((NHEADS * NHID, NHEADS), lambda i: (0, 0)),
            pl.BlockSpec((NHEADS * NHID, NHEADS), lambda i: (0, 0)),
        ],
        out_specs=[
            pl.BlockSpec((BI, 16), lambda i: (i, 0)),
            pl.BlockSpec((BI, 128), lambda i: (i, 0)),
        ],
        out_shape=[
            jax.ShapeDtypeStruct((NPAD, 16), f32),
            jax.ShapeDtypeStruct((N, 128), f32),
        ],
    )(x, wc, a1, a2)

    utab_pad = utab.reshape(NPAD * 16)

    mesh = plsc.VectorSubcoreMesh(core_axis_name="c", subcore_axis_name="s")
    sk1 = pl.kernel(
        _sk1_body, mesh=mesh,
        out_type=[
            jax.ShapeDtypeStruct((NPAD * 80,), f32),
            jax.ShapeDtypeStruct((NWORKERS * EMAX,), jnp.int32),
            jax.ShapeDtypeStruct((NWORKERS * EMAX,), jnp.int32),
            jax.ShapeDtypeStruct((NWORKERS * 16,), jnp.int32),
        ],
        scratch_types=[
            pltpu.VMEM((N,), jnp.int32),
            pltpu.VMEM((EMAX,), jnp.int32),
            pltpu.VMEM((EMAX,), jnp.int32),
            pltpu.VMEM((ROWS_W * 16,), f32),
            pltpu.VMEM((ROWS_W * 80,), f32),
            pltpu.VMEM((G, 128), f32),
            pltpu.VMEM((16,), jnp.int32),
            pltpu.SemaphoreType.DMA,
        ],
        compiler_params=pltpu.CompilerParams(needs_layout_passes=False),
    )
    acc1, srcL, dstL, cnts = sk1(mask_flat, utab_pad, rec1)

    wml = jnp.concatenate([W_mu, W_lv], axis=1)
    z2 = jnp.zeros((NOUT, 1), f32)
    b1 = jnp.concatenate([
        jnp.concatenate([a_mu[0, :NOUT, None], z2], axis=1),
        jnp.concatenate([z2, a_lv[0, :NOUT, None]], axis=1)], axis=0)
    b2 = jnp.concatenate([
        jnp.concatenate([a_mu[0, NOUT:, None], z2], axis=1),
        jnp.concatenate([z2, a_lv[0, NOUT:, None]], axis=1)], axis=0)

    utab2, rec2 = pl.pallas_call(
        _k3_body,
        grid=(NI,),
        in_specs=[
            pl.BlockSpec((BI, 80), lambda i: (i, 0)),
            pl.BlockSpec((NHEADS * NHID, 2 * NOUT), lambda i: (0, 0)),
            pl.BlockSpec((2 * NOUT, 2), lambda i: (0, 0)),
            pl.BlockSpec((2 * NOUT, 2), lambda i: (0, 0)),
        ],
        out_specs=[
            pl.BlockSpec((BI, 16), lambda i: (i, 0)),
            pl.BlockSpec((BI, 128), lambda i: (i, 0)),
        ],
        out_shape=[
            jax.ShapeDtypeStruct((NPAD, 16), f32),
            jax.ShapeDtypeStruct((N, 128), f32),
        ],
    )(acc1.reshape(NPAD, 80), wml, b1, b2)

    utab2_pad = utab2.reshape(NPAD * 16)

    sk2 = pl.kernel(
        _sk2_body, mesh=mesh,
        out_type=[jax.ShapeDtypeStruct((NPAD * 48,), f32)],
        scratch_types=[
            pltpu.VMEM((EMAX,), jnp.int32),
            pltpu.VMEM((EMAX,), jnp.int32),
            pltpu.VMEM((ROWS_W * 16,), f32),
            pltpu.VMEM((ROWS_W * 48,), f32),
            pltpu.VMEM((G, 128), f32),
            pltpu.VMEM((NWORKERS * 16,), jnp.int32),
            pltpu.SemaphoreType.DMA,
        ],
        compiler_params=pltpu.CompilerParams(needs_layout_passes=False),
    )
    acc2, = sk2(utab2_pad, rec2, srcL, dstL, cnts)

    mu, lv = pl.pallas_call(
        _k5_body,
        grid=(NI,),
        in_specs=[pl.BlockSpec((BI, 48), lambda i: (i, 0))],
        out_specs=[
            pl.BlockSpec((BI, NOUT), lambda i: (i, 0)),
            pl.BlockSpec((BI, NOUT), lambda i: (i, 0)),
        ],
        out_shape=[
            jax.ShapeDtypeStruct((N, NOUT), f32),
            jax.ShapeDtypeStruct((N, NOUT), f32),
        ],
    )(acc2.reshape(NPAD, 48))

    return (mu, mu, lv)


# speedup vs baseline: 1.4826x; 1.0786x over previous
import jax
import jax.numpy as jnp
from jax import lax
from jax.experimental import pallas as pl
from jax.experimental.pallas import tpu as pltpu
from jax.experimental.pallas import tpu_sc as plsc

N = 10000
NFEAT = 128
NHID = 8
NOUT = 16
NHEADS = 8
ALPHA = 0.2

R = 20
NWROW = N // R
NPAD = 10400
BI = 400
NI = N // BI

NWORKERS = 32
WPW = 16
ROWS_W = WPW * R
EMAX = 16384
G = 128


def _take16(x, idx):
    dn = lax.GatherDimensionNumbers(
        offset_dims=(), collapsed_slice_dims=(0,), start_index_map=(0,))
    return lax.gather(x, idx[:, None], dn, (1,),
                      mode=lax.GatherScatterMode.PROMISE_IN_BOUNDS)

def _kp_body(adj_ref, out_ref):
    a3 = adj_ref[...].astype(jnp.int32).reshape(2, R, N)
    r = lax.broadcasted_iota(jnp.int32, (2, R, N), 1)
    out_ref[...] = jnp.sum(a3 << r, axis=1).reshape(1, 2, N)


def _k1_body(x_ref, wc_ref, a1_ref, a2_ref, utab_ref, rec1_ref):
    h = jnp.dot(x_ref[...], wc_ref[...], preferred_element_type=jnp.float32)
    s1 = jnp.dot(h, a1_ref[...], preferred_element_type=jnp.float32)
    s2 = jnp.dot(h, a2_ref[...], preferred_element_type=jnp.float32)
    utab_ref[...] = jnp.concatenate([jnp.exp(-s1), jnp.exp(-ALPHA * s1)], 1)
    zp = jnp.zeros((h.shape[0], 48), jnp.float32)
    rec1_ref[...] = jnp.concatenate(
        [jnp.exp(-s2), jnp.exp(-ALPHA * s2), h, zp], 1)


def _sk1_body(mask_hbm,
              src_hbm, dst_hbm, cnt_hbm,
              maskb, srcl, dstl, cntb):
    wid = lax.axis_index("s") * 2 + lax.axis_index("c")
    wstart = wid * WPW
    wcnt = jnp.minimum(WPW, NWROW - wstart)

    iota = lax.iota(jnp.int32, 16)
    z16i = jnp.zeros((16,), jnp.int32)

    def _zero_d(i, carry):
        dstl[pl.ds(i * 16, 16)] = z16i
        return carry

    lax.fori_loop(0, EMAX // 16, _zero_d, 0)

    def _row_loop(rl, cur):
        w = wstart + rl
        pltpu.sync_copy(mask_hbm.at[pl.ds(w * N, N)], maskb)
        srcbase = w * R

        def _chunk_loop(ch, cur):
            words = maskb[pl.ds(ch * 16, 16)]
            nzm0 = (words != 0).astype(jnp.int32)

            def _cond(st):
                return jnp.any(st[1] != 0)

            def _word(st):
                cur2, nzm = st
                lanev = plsc.all_reduce_ffs(nzm != 0)
                if getattr(lanev, "ndim", 0) == 0:
                    lanev = jnp.full((16,), lanev, jnp.int32)
                wb = _take16(words, lanev)
                dstv = ch * 16 + lanev
                m1 = ((wb >> iota) & 1) == 1
                c1 = jnp.cumsum(m1.astype(jnp.int32))
                pos1 = cur2 + c1 - 1
                ok1 = m1 & (pos1 < EMAX)
                plsc.store_scatter(srcl, [pos1], srcbase + iota, mask=ok1)
                plsc.store_scatter(dstl, [pos1], dstv, mask=ok1)
                n1 = jnp.sum(ok1.astype(jnp.int32))
                m2 = ((((wb >> (iota + 16)) & 1) == 1) & (iota < 4))
                c2 = jnp.cumsum(m2.astype(jnp.int32))
                pos2 = cur2 + n1 + c2 - 1
                ok2 = m2 & (pos2 < EMAX)
                plsc.store_scatter(srcl, [pos2], srcbase + 16 + iota, mask=ok2)
                plsc.store_scatter(dstl, [pos2], dstv, mask=ok2)
                n2 = jnp.sum(ok2.astype(jnp.int32))
                nzm2 = jnp.where(iota == lanev, 0, nzm)
                return (cur2 + n1 + n2, nzm2)

            cur, _ = lax.while_loop(_cond, _word, (cur, nzm0))
            return cur

        return lax.fori_loop(0, N // 16, _chunk_loop, cur)

    cnt = lax.fori_loop(0, wcnt, _row_loop, 0)

    pltpu.sync_copy(srcl, src_hbm.at[pl.ds(wid * EMAX, EMAX)])
    pltpu.sync_copy(dstl, dst_hbm.at[pl.ds(wid * EMAX, EMAX)])
    cntb[pl.ds(0, 16)] = jnp.full((16,), cnt, jnp.int32)
    pltpu.sync_copy(cntb, cnt_hbm.at[pl.ds(wid * 16, 16)])


BIA = 80
NIA = N // BIA


def _att_body(adj_ref, u_ref, ua_ref, vt_ref, vat_ref, haug_ref, out_ref):
    adjf = adj_ref[...].astype(jnp.float32)
    outs = []
    for k in range(NHEADS):
        p1 = u_ref[:, k:k + 1] * vt_ref[k:k + 1, :]
        p2 = ua_ref[:, k:k + 1] * vat_ref[k:k + 1, :]
        e = jnp.minimum(p1, p2) * adjf
        outs.append(jnp.dot(e, haug_ref[:, k * 9:(k + 1) * 9],
                            preferred_element_type=jnp.float32))
    out_ref[...] = jnp.concatenate(outs, axis=1)


def _k3_body(acc_ref, wml_ref, b1_ref, b2_ref, utab2_ref, rec2_ref):
    acc = acc_ref[...]
    hs = []
    for k in range(NHEADS):
        num = acc[:, k * 9:k * 9 + NHID]
        den = acc[:, k * 9 + NHID:k * 9 + NHID + 1]
        hp = num / den
        hs.append(jnp.where(hp > 0, hp, jnp.exp(hp) - 1.0))
    h1 = jnp.concatenate(hs, axis=1)
    g = jnp.dot(h1, wml_ref[...], preferred_element_type=jnp.float32)
    s1 = jnp.dot(g, b1_ref[...], preferred_element_type=jnp.float32)
    s2 = jnp.dot(g, b2_ref[...], preferred_element_type=jnp.float32)
    zpad = jnp.zeros((acc.shape[0], 12), jnp.float32)
    utab2_ref[...] = jnp.concatenate(
        [jnp.exp(-s1), jnp.exp(-ALPHA * s1), zpad], 1)
    zp48 = jnp.zeros((acc.shape[0], 80), jnp.float32)
    rec2_ref[...] = jnp.concatenate(
        [jnp.exp(-s2), jnp.exp(-ALPHA * s2), zpad, g, zp48], 1)


def _sk2_body(utab_hbm, rec2_hbm, src_hbm, dst_hbm, cnt_hbm,
              acc_hbm, srcl, dstl, utabl, accl, gbuf, cntb, sem):
    wid = lax.axis_index("s") * 2 + lax.axis_index("c")
    wstart = wid * WPW
    rowbase = wstart * R

    iota = lax.iota(jnp.int32, 16)
    z16f = jnp.zeros((16,), jnp.float32)

    def _zero_a(i, carry):
        accl[pl.ds(i * 16, 16)] = z16f
        return carry

    lax.fori_loop(0, ROWS_W * 48 // 16, _zero_a, 0)

    pltpu.sync_copy(utab_hbm.at[pl.ds(rowbase * 16, ROWS_W * 16)], utabl)
    pltpu.sync_copy(src_hbm.at[pl.ds(wid * EMAX, EMAX)], srcl)
    pltpu.sync_copy(dst_hbm.at[pl.ds(wid * EMAX, EMAX)], dstl)
    pltpu.sync_copy(cnt_hbm, cntb)
    cnt = cntb[pl.ds(wid * 16, 16)][0]

    nb = (cnt + G - 1) // G

    def _batch(b, carry):
        idx = dstl.at[pl.ds(b * G, G)]
        pltpu.async_copy(rec2_hbm.at[idx], gbuf, sem).wait()
        rem = jnp.minimum(G, cnt - b * G)

        def _group(g, carry2):
            srcv = srcl[pl.ds(b * G + g * 16, 16)]
            gcnt = jnp.clip(rem - g * 16, 0, 16)

            def _edge(e16, carry3):
                sl = jnp.sum(jnp.where(iota == e16, srcv, 0)) - rowbase
                e = g * 16 + e16
                uvec = utabl[pl.ds(sl * 16, 16)]
                rv = gbuf[e, pl.ds(0, 16)]
                prod = uvec * rv
                psh = _take16(prod, (iota + 2) & 15)
                ev = jnp.minimum(prod, psh)
                rs = jnp.where(iota < 2, ev, 0.0)
                plsc.addupdate(accl.at[pl.ds(sl * 48, 16)], rs)
                emu = _take16(ev, jnp.zeros((16,), jnp.int32))
                elv = _take16(ev, jnp.ones((16,), jnp.int32))
                gmu = gbuf[e, pl.ds(16, 16)]
                glv = gbuf[e, pl.ds(32, 16)]
                plsc.addupdate(accl.at[pl.ds(sl * 48 + 16, 16)], emu * gmu)
                plsc.addupdate(accl.at[pl.ds(sl * 48 + 32, 16)], elv * glv)
                return carry3

            lax.fori_loop(0, gcnt, _edge, 0)
            return carry2

        lax.fori_loop(0, G // 16, _group, 0)
        return carry

    lax.fori_loop(0, nb, _batch, 0)

    pltpu.sync_copy(accl, acc_hbm.at[pl.ds(rowbase * 48, ROWS_W * 48)])


def _k5_body(acc_ref, mu_ref, lv_ref):
    acc = acc_ref[...]
    mu_ref[...] = acc[:, 16:32] / acc[:, 0:1]
    lv_ref[...] = acc[:, 32:48] / acc[:, 1:2]


def kernel(x, adj, W, a, W_mu, a_mu, W_lv, a_lv):
    f32 = jnp.float32

    wc = jnp.transpose(W, (1, 0, 2)).reshape(NFEAT, NHEADS * NHID)
    eye = jnp.eye(NHEADS, dtype=f32)
    a1 = (a[:, 0, :NHID][:, :, None] * eye[:, None, :]).reshape(
        NHEADS * NHID, NHEADS)
    a2 = (a[:, 0, NHID:][:, :, None] * eye[:, None, :]).reshape(
        NHEADS * NHID, NHEADS)

    colmask = pl.pallas_call(
        _kp_body,
        grid=(NWROW // 2,),
        in_specs=[pl.BlockSpec((2 * R, N), lambda i: (i, 0))],
        out_specs=pl.BlockSpec((1, 2, N), lambda i: (i, 0, 0)),
        out_shape=jax.ShapeDtypeStruct((NWROW // 2, 2, N), jnp.int32),
    )(adj)
    mask_flat = colmask.reshape(NWROW * N)

    utab, rec1 = pl.pallas_call(
        _k1_body,
        grid=(NI,),
        in_specs=[
            pl.BlockSpec((BI, NFEAT), lambda i: (i, 0)),
            pl.BlockSpec((NFEAT, NHEADS * NHID), lambda i: (0, 0)),
            pl.BlockSpec((NHEADS * NHID, NHEADS), lambda i: (0, 0)),
            pl.BlockSpec((NHEADS * NHID, NHEADS), lambda i: (0, 0)),
        ],
        out_specs=[
            pl.BlockSpec((BI, 16), lambda i: (i, 0)),
            pl.BlockSpec((BI, 128), lambda i: (i, 0)),
        ],
        out_shape=[
            jax.ShapeDtypeStruct((NPAD, 16), f32),
            jax.ShapeDtypeStruct((N, 128), f32),
        ],
    )(x, wc, a1, a2)

    utab_pad = utab.reshape(NPAD * 16)

    mesh = plsc.VectorSubcoreMesh(core_axis_name="c", subcore_axis_name="s")
    sk1 = pl.kernel(
        _sk1_body, mesh=mesh,
        out_type=[
            jax.ShapeDtypeStruct((NWORKERS * EMAX,), jnp.int32),
            jax.ShapeDtypeStruct((NWORKERS * EMAX,), jnp.int32),
            jax.ShapeDtypeStruct((NWORKERS * 16,), jnp.int32),
        ],
        scratch_types=[
            pltpu.VMEM((N,), jnp.int32),
            pltpu.VMEM((EMAX,), jnp.int32),
            pltpu.VMEM((EMAX,), jnp.int32),
            pltpu.VMEM((16,), jnp.int32),
        ],
        compiler_params=pltpu.CompilerParams(needs_layout_passes=False),
    )
    srcL, dstL, cnts = sk1(mask_flat)

    u = utab[:N, 0:8]
    ua = utab[:N, 8:16]
    vt = rec1[:, 0:8].T
    vat = rec1[:, 8:16].T
    ones1 = jnp.ones((N, 1), f32)
    haug = jnp.concatenate(
        [jnp.concatenate([rec1[:, 16 + k * 8:16 + (k + 1) * 8], ones1], 1)
         for k in range(NHEADS)], 1)
    acc1 = pl.pallas_call(
        _att_body,
        grid=(NIA,),
        in_specs=[
            pl.BlockSpec((BIA, N), lambda i: (i, 0)),
            pl.BlockSpec((BIA, NHEADS), lambda i: (i, 0)),
            pl.BlockSpec((BIA, NHEADS), lambda i: (i, 0)),
            pl.BlockSpec((NHEADS, N), lambda i: (0, 0)),
            pl.BlockSpec((NHEADS, N), lambda i: (0, 0)),
            pl.BlockSpec((N, NHEADS * 9), lambda i: (0, 0)),
        ],
        out_specs=pl.BlockSpec((BIA, NHEADS * 9), lambda i: (i, 0)),
        out_shape=jax.ShapeDtypeStruct((N, NHEADS * 9), f32),
    )(adj, u, ua, vt, vat, haug)

    wml = jnp.concatenate([W_mu, W_lv], axis=1)
    z2 = jnp.zeros((NOUT, 1), f32)
    b1 = jnp.concatenate([
        jnp.concatenate([a_mu[0, :NOUT, None], z2], axis=1),
        jnp.concatenate([z2, a_lv[0, :NOUT, None]], axis=1)], axis=0)
    b2 = jnp.concatenate([
        jnp.concatenate([a_mu[0, NOUT:, None], z2], axis=1),
        jnp.concatenate([z2, a_lv[0, NOUT:, None]], axis=1)], axis=0)

    utab2, rec2 = pl.pallas_call(
        _k3_body,
        grid=(NI,),
        in_specs=[
            pl.BlockSpec((BI, NHEADS * 9), lambda i: (i, 0)),
            pl.BlockSpec((NHEADS * NHID, 2 * NOUT), lambda i: (0, 0)),
            pl.BlockSpec((2 * NOUT, 2), lambda i: (0, 0)),
            pl.BlockSpec((2 * NOUT, 2), lambda i: (0, 0)),
        ],
        out_specs=[
            pl.BlockSpec((BI, 16), lambda i: (i, 0)),
            pl.BlockSpec((BI, 128), lambda i: (i, 0)),
        ],
        out_shape=[
            jax.ShapeDtypeStruct((NPAD, 16), f32),
            jax.ShapeDtypeStruct((N, 128), f32),
        ],
    )(acc1, wml, b1, b2)

    utab2_pad = utab2.reshape(NPAD * 16)

    sk2 = pl.kernel(
        _sk2_body, mesh=mesh,
        out_type=[jax.ShapeDtypeStruct((NPAD * 48,), f32)],
        scratch_types=[
            pltpu.VMEM((EMAX,), jnp.int32),
            pltpu.VMEM((EMAX,), jnp.int32),
            pltpu.VMEM((ROWS_W * 16,), f32),
            pltpu.VMEM((ROWS_W * 48,), f32),
            pltpu.VMEM((G, 128), f32),
            pltpu.VMEM((NWORKERS * 16,), jnp.int32),
            pltpu.SemaphoreType.DMA,
        ],
        compiler_params=pltpu.CompilerParams(needs_layout_passes=False),
    )
    acc2, = sk2(utab2_pad, rec2, srcL, dstL, cnts)

    mu, lv = pl.pallas_call(
        _k5_body,
        grid=(NI,),
        in_specs=[pl.BlockSpec((BI, 48), lambda i: (i, 0))],
        out_specs=[
            pl.BlockSpec((BI, NOUT), lambda i: (i, 0)),
            pl.BlockSpec((BI, NOUT), lambda i: (i, 0)),
        ],
        out_shape=[
            jax.ShapeDtypeStruct((N, NOUT), f32),
            jax.ShapeDtypeStruct((N, NOUT), f32),
        ],
    )(acc2.reshape(NPAD, 48))

    return (mu, mu, lv)
